# Initial kernel scaffold; baseline (speedup 1.0000x reference)
import jax
import jax.numpy as jnp
from jax.experimental import pallas as pl


def kernel(emb_q, emb_q2, emb_s, emb_u, emb_r, w1_q, w2_q, W_ih, W_hh, b_ih, b_hh, fusion_W, fusion_b, agg_W, agg_b, aggL_W, aggL_b, q_W, q_b, k_W, k_b, w_W, w_b, user, question, response, mask, q_neighbors, s_neighbors, u_neighbors, q_neighbors_2, q_skill_idx):
    raise NotImplementedError("write your pallas kernel here")


# speedup vs baseline: 6.0873x; 6.0873x over previous
import functools

import jax
import jax.numpy as jnp
from jax import lax
from jax.experimental import pallas as pl
from jax.experimental.pallas import tpu as pltpu
from jax.experimental.pallas import tpu_sc as plsc

NQ, NS, NU = 10000, 1000, 20000
D, B, T = 100, 64, 50
NB = 4
RANK_K = 10
MAX_S = 4
DP = 128
Tm1 = T - 1
N_REAL = Tm1 * B
NWK = 32
TW = 104
NT = NWK * TW
TP = 56
NEG = float(-3.0e38)


def _sc_gather_fn(qflat, uflat, qn_tbl, sn_tbl, un_tbl, qn2_tbl, qsk_tbl,
                  embq, embs, embu, embq2,
                  E0, E1, E2, E3, F0, F1, F2, F3, QN, SK,
                  qts, uts, qnx,
                  x4a, n1_f, x4b, n2_f, x4c, n3_f, sk_f, R,
                  s0, s1, s2, s3):
    sems = (s0, s1, s2, s3)
    wid = lax.axis_index("s") * 2 + lax.axis_index("c")
    base = wid * TW

    def take16(v, idx):
        dn = lax.GatherDimensionNumbers(offset_dims=(),
                                        collapsed_slice_dims=(0,),
                                        start_index_map=(0,))
        return lax.gather(v, idx[:, None], dn, slice_sizes=(1,),
                          mode=lax.GatherScatterMode.PROMISE_IN_BOUNDS)

    def expand4(src1d, dst1d, nchunks2):
        def body(c, carry):
            it = lax.iota(jnp.int32, 16)
            sub = lax.shift_right_logical(it, 2)
            cl = lax.bitwise_and(it, 3)
            v = src1d[pl.ds(c * 8, 16)]
            a = take16(v, sub)
            bvals = take16(v, sub + 4)
            dst1d[pl.ds(c * 32, 16)] = a * 4 + cl
            dst1d[pl.ds(c * 32 + 16, 16)] = bvals * 4 + cl
            return carry
        lax.fori_loop(0, nchunks2, body, 0)

    def elem_level(flat_tbl, idx_ref, nparts, dst1d):
        for g0 in range(0, nparts, 4):
            gcnt = min(4, nparts - g0)
            hs = []
            for j in range(gcnt):
                p = g0 + j
                hs.append(pltpu.async_copy(
                    flat_tbl.at[idx_ref.at[pl.ds(p * TW, TW)]],
                    dst1d.at[pl.ds(p * TW, TW)], sems[j]))
            for h in hs:
                h.wait()

    def row_level(emb_tbl, idx_ref, nparts, out_hbm, out_base):
        for g0 in range(0, nparts, 4):
            gcnt = min(4, nparts - g0)
            hs = []
            for j in range(gcnt):
                p = g0 + j
                hs.append(pltpu.async_copy(
                    emb_tbl.at[idx_ref.at[pl.ds(p * TW, TW)]],
                    R.at[pl.ds(j * TW, TW)], sems[j]))
            for h in hs:
                h.wait()
            pltpu.sync_copy(R.at[pl.ds(0, gcnt * TW)],
                            out_hbm.at[pl.ds(out_base + g0 * TW, gcnt * TW)])

    def tree(idx0_ref, hop1_flat, hop2_flat, emb_even, emb_odd,
             O0, O1, O2, O3):
        row_level(emb_even, idx0_ref, 1, O0, base)
        expand4(idx0_ref, x4a, 13)
        elem_level(hop1_flat, x4a, 4, n1_f)
        row_level(emb_odd, n1_f, 4, O1, 4 * base)
        expand4(n1_f, x4b, 52)
        elem_level(hop2_flat, x4b, 16, n2_f)
        row_level(emb_even, n2_f, 16, O2, 16 * base)
        expand4(n2_f, x4c, 208)
        elem_level(hop1_flat, x4c, 64, n3_f)
        row_level(emb_odd, n3_f, 64, O3, 64 * base)

    pltpu.sync_copy(qflat.at[pl.ds(base, TW + 8)], qts)
    pltpu.sync_copy(uflat.at[pl.ds(base, TW + 8)], uts)
    pltpu.sync_copy(qflat.at[pl.ds(base + B, TW + 8)], qnx)

    tree(qts, qn_tbl, sn_tbl, embq, embs, E0, E1, E2, E3)
    tree(uts, un_tbl, qn2_tbl, embu, embq2, F0, F1, F2, F3)

    row_level(embq, qnx, 1, QN, base)
    expand4(qnx, x4a, 13)
    elem_level(qsk_tbl, x4a, 4, sk_f)
    row_level(embs, sk_f, 4, SK, 4 * base)


def _sc_gather(qflat, uflat, qn_tbl, sn_tbl, un_tbl, qn2_tbl, qsk_tbl,
               embq, embs, embu, embq2):
    f32, i32 = jnp.float32, jnp.int32
    out_type = [
        jax.ShapeDtypeStruct((NT, DP), f32),
        jax.ShapeDtypeStruct((NT * 4, DP), f32),
        jax.ShapeDtypeStruct((NT * 16, DP), f32),
        jax.ShapeDtypeStruct((NT * 64, DP), f32),
        jax.ShapeDtypeStruct((NT, DP), f32),
        jax.ShapeDtypeStruct((NT * 4, DP), f32),
        jax.ShapeDtypeStruct((NT * 16, DP), f32),
        jax.ShapeDtypeStruct((NT * 64, DP), f32),
        jax.ShapeDtypeStruct((NT, DP), f32),
        jax.ShapeDtypeStruct((NT * 4, DP), f32),
    ]
    scratch = [
        pltpu.VMEM((TW + 8,), i32), pltpu.VMEM((TW + 8,), i32),
        pltpu.VMEM((TW + 8,), i32),
        pltpu.VMEM((4 * TW,), i32), pltpu.VMEM((4 * TW + 32,), i32),
        pltpu.VMEM((16 * TW,), i32), pltpu.VMEM((16 * TW + 32,), i32),
        pltpu.VMEM((64 * TW,), i32), pltpu.VMEM((64 * TW,), i32),
        pltpu.VMEM((4 * TW,), i32),
        pltpu.VMEM((4 * TW, DP), f32),
        pltpu.SemaphoreType.DMA, pltpu.SemaphoreType.DMA,
        pltpu.SemaphoreType.DMA, pltpu.SemaphoreType.DMA,
    ]
    mesh = plsc.VectorSubcoreMesh(core_axis_name="c", subcore_axis_name="s")
    return pl.kernel(_sc_gather_fn, mesh=mesh, out_type=out_type,
                     scratch_types=scratch)(
        qflat, uflat, qn_tbl, sn_tbl, un_tbl, qn2_tbl, qsk_tbl,
        embq, embs, embu, embq2)


TB = 64
G1 = NT // TB


def _k1_fn(e0, e1f, e1g, e2f, e2g, e3g,
           f0, f1f, f1g, f2f, f2g, f3g,
           qn, skg, er2,
           W0, W1, W2, WL, Fw1, Wih, P1, P2, bp, xb, wv, qvec,
           xp_out, un_out):
    r = jax.nn.relu

    def dot(a, b):
        return jnp.dot(a, b, preferred_element_type=jnp.float32)

    b0 = bp[0:1, :]
    b1 = bp[1:2, :]
    b2 = bp[2:3, :]
    bL = bp[3:4, :]

    def tree(x0, x1f, x1g, x2f, x2g, x3g):
        m3 = jnp.mean(x3g[...], axis=1)
        A2 = r(dot(m3 + x2f[...], W2[...]) + b2)
        A1 = r(dot(jnp.mean(x2g[...], axis=1) + x1f[...], W1[...]) + b1)
        A0 = r(dot(jnp.mean(x1g[...], axis=1) + x0[...], W0[...]) + b0)
        B0 = r(dot(dot(P1[...], A1) + A0, W0[...]) + b0)
        B1 = r(dot(dot(P2[...], A2) + A1, W1[...]) + b1)
        C0 = r(dot(dot(P1[...], B1) + B0, W0[...]) + b0)
        return r(dot(C0, WL[...]) + bL)

    g1 = tree(e0, e1f, e1g, e2f, e2g, e3g)
    g2 = tree(f0, f1f, f1g, f2f, f2g, f3g)
    ehat = g1 * wv[0:1, :] + g2 * wv[1:2, :]
    e_t = r(dot(ehat, Fw1[...]) + er2[...])
    xp_out[...] = dot(e_t, Wih[...]) + xb[...]

    qnv = qn[...]
    skv = skg[...]
    qv = qvec[...]
    qd0 = jnp.sum(qnv * qv, axis=-1, keepdims=True)
    qds = jnp.sum(skv * qv[None], axis=-1)
    qall = jnp.concatenate([qd0, qds], axis=1)
    mq = jnp.max(qall, axis=1, keepdims=True)
    wq = jnp.exp(qall - mq)
    u = wq[:, 0:1] * qnv
    for j in range(MAX_S):
        u = u + wq[:, j + 1:j + 2] * skv[:, j, :]
    sq = jnp.sum(wq, axis=1, keepdims=True)
    un_out[...] = u / sq


def _k1(e0, e1, e2, e3, f0, f1, f2, f3, qn, sk, er2,
        W0, W1, W2, WL, Fw1, Wih, P1, P2, bp, xb, wv, qvec):
    f32 = jnp.float32
    full = lambda shape: pl.BlockSpec(shape, lambda i: tuple(0 for _ in shape))
    specs = [
        pl.BlockSpec((TB, DP), lambda i: (i, 0)),
        pl.BlockSpec((4 * TB, DP), lambda i: (i, 0)),
        pl.BlockSpec((TB, 4, DP), lambda i: (i, 0, 0)),
        pl.BlockSpec((16 * TB, DP), lambda i: (i, 0)),
        pl.BlockSpec((4 * TB, 4, DP), lambda i: (i, 0, 0)),
        pl.BlockSpec((16 * TB, 4, DP), lambda i: (i, 0, 0)),
        pl.BlockSpec((TB, DP), lambda i: (i, 0)),
        pl.BlockSpec((4 * TB, DP), lambda i: (i, 0)),
        pl.BlockSpec((TB, 4, DP), lambda i: (i, 0, 0)),
        pl.BlockSpec((16 * TB, DP), lambda i: (i, 0)),
        pl.BlockSpec((4 * TB, 4, DP), lambda i: (i, 0, 0)),
        pl.BlockSpec((16 * TB, 4, DP), lambda i: (i, 0, 0)),
        pl.BlockSpec((TB, DP), lambda i: (i, 0)),
        pl.BlockSpec((TB, 4, DP), lambda i: (i, 0, 0)),
        pl.BlockSpec((TB, DP), lambda i: (i, 0)),
        full((DP, DP)), full((DP, DP)), full((DP, DP)), full((DP, DP)),
        full((DP, DP)), full((DP, 4 * DP)),
        full((TB, 4 * TB)), full((4 * TB, 16 * TB)),
        full((8, DP)), full((1, 4 * DP)), full((2, DP)), full((1, DP)),
    ]
    return pl.pallas_call(
        _k1_fn,
        grid=(G1,),
        in_specs=specs,
        out_specs=[pl.BlockSpec((TB, 4 * DP), lambda i: (i, 0)),
                   pl.BlockSpec((TB, DP), lambda i: (i, 0))],
        out_shape=[jax.ShapeDtypeStruct((NT, 4 * DP), f32),
                   jax.ShapeDtypeStruct((NT, DP), f32)],
    )(e0, e1, e1.reshape(NT, 4, DP), e2, e2.reshape(NT * 4, 4, DP),
      e3.reshape(NT * 16, 4, DP), f0, f1, f1.reshape(NT, 4, DP),
      f2, f2.reshape(NT * 4, 4, DP), f3.reshape(NT * 16, 4, DP),
      qn, sk.reshape(NT, 4, DP), er2,
      W0, W1, W2, WL, Fw1, Wih, P1, P2, bp, xb, wv, qvec)


def _k2_fn(xp_ref, whh, hist_out, h, c):
    t = pl.program_id(0)

    @pl.when(t == 0)
    def _():
        h[...] = jnp.zeros((B, DP), jnp.float32)
        c[...] = jnp.zeros((B, DP), jnp.float32)

    g = xp_ref[0] + jnp.dot(h[...], whh[...],
                            preferred_element_type=jnp.float32)
    i_g = g[:, 0:DP]
    f_g = g[:, DP:2 * DP]
    g_g = g[:, 2 * DP:3 * DP]
    o_g = g[:, 3 * DP:4 * DP]
    c2 = jax.nn.sigmoid(f_g) * c[...] + jax.nn.sigmoid(i_g) * jnp.tanh(g_g)
    h2 = jax.nn.sigmoid(o_g) * jnp.tanh(c2)
    h[...] = h2
    c[...] = c2
    hist_out[0] = h2


def _k2(xproj, whh):
    return pl.pallas_call(
        _k2_fn,
        grid=(Tm1,),
        in_specs=[pl.BlockSpec((1, B, 4 * DP), lambda t: (t, 0, 0)),
                  pl.BlockSpec((DP, 4 * DP), lambda t: (0, 0))],
        out_specs=pl.BlockSpec((1, B, DP), lambda t: (t, 0, 0)),
        out_shape=jax.ShapeDtypeStruct((Tm1, B, DP), jnp.float32),
        scratch_shapes=[pltpu.VMEM((B, DP), jnp.float32),
                        pltpu.VMEM((B, DP), jnp.float32)],
    )(xproj, whh)


def _k3_fn(gq_ref, gs_ref, h_ref, u_ref, kv_ref, out_ref):
    Gv = gq_ref[0]
    Gs = gs_ref[0]
    H = h_ref[0]
    U = u_ref[0]
    dims = (((1,), (1,)), ((), ()))
    Srow = lax.dot_general(Gs, Gv, dims,
                           preferred_element_type=jnp.float32)
    tt = lax.broadcasted_iota(jnp.int32, (TP, TP), 0)
    ta = lax.broadcasted_iota(jnp.int32, (TP, TP), 1)
    valid = (ta < tt) & (tt < Tm1)
    Sm = jnp.where(valid, Srow, NEG)
    gtr = jnp.sum((Sm[:, :, None] > Sm[:, None, :]).astype(jnp.int32),
                  axis=1)
    i1 = lax.broadcasted_iota(jnp.int32, (TP, TP, TP), 1)
    i2 = lax.broadcasted_iota(jnp.int32, (TP, TP, TP), 2)
    eqc = jnp.sum(((Sm[:, :, None] == Sm[:, None, :]) & (i1 < i2))
                  .astype(jnp.int32), axis=1)
    rank = gtr + eqc
    sel = (valid & (rank < RANK_K)) | (valid & (tt < RANK_K)) \
        | ((ta == tt) & (tt < Tm1))
    hk = lax.dot_general(kv_ref[...], H, dims,
                         preferred_element_type=jnp.float32)
    hkb = jnp.broadcast_to(hk, (TP, TP))
    mk = jnp.max(jnp.where(sel, hkb, NEG), axis=1, keepdims=True)
    wk = jnp.where(sel, jnp.exp(hkb - mk), 0.0)
    sk = jnp.sum(wk, axis=1, keepdims=True)
    V = jnp.dot(wk, H, preferred_element_type=jnp.float32)
    num = jnp.sum(U * V, axis=1, keepdims=True)
    p = num / sk
    out_ref[0] = jnp.broadcast_to(p, (TP, DP))


def _k3(gqb, gsb, histb, unb, kvec):
    spec = pl.BlockSpec((1, TP, DP), lambda b: (b, 0, 0))
    return pl.pallas_call(
        _k3_fn,
        grid=(B,),
        in_specs=[spec, spec, spec, spec,
                  pl.BlockSpec((1, DP), lambda b: (0, 0))],
        out_specs=spec,
        out_shape=jax.ShapeDtypeStruct((B, TP, DP), jnp.float32),
    )(gqb, gsb, histb, unb, kvec)


def kernel(emb_q, emb_q2, emb_s, emb_u, emb_r, w1_q, w2_q, W_ih, W_hh,
           b_ih, b_hh, fusion_W, fusion_b, agg_W, agg_b, aggL_W, aggL_b,
           q_W, q_b, k_W, k_b, w_W, w_b, user, question, response, mask,
           q_neighbors, s_neighbors, u_neighbors, q_neighbors_2,
           q_skill_idx):
    f32 = jnp.float32
    padD = lambda x: jnp.pad(x, ((0, 0), (0, DP - D)))
    embq_p = padD(emb_q)
    embs_p = padD(emb_s)
    embu_p = padD(emb_u)
    embq2_p = padD(emb_q2)

    qflat = jnp.pad(question.T.reshape(-1).astype(jnp.int32),
                    (0, 3520 - T * B))
    uflat = jnp.pad(user.T.reshape(-1).astype(jnp.int32), (0, 3520 - T * B))

    outs = _sc_gather(qflat, uflat,
                      q_neighbors.astype(jnp.int32).reshape(-1),
                      s_neighbors.astype(jnp.int32).reshape(-1),
                      u_neighbors.astype(jnp.int32).reshape(-1),
                      q_neighbors_2.astype(jnp.int32).reshape(-1),
                      q_skill_idx.astype(jnp.int32).reshape(-1),
                      embq_p, embs_p, embu_p, embq2_p)
    E0, E1, E2, E3, F0, F1, F2, F3, QN, SK = outs

    padW = lambda w: jnp.pad(w, ((0, DP - w.shape[0]), (0, DP - w.shape[1])))
    W0, W1, W2 = padW(agg_W[0]), padW(agg_W[1]), padW(agg_W[2])
    WL = padW(aggL_W)
    Fw1 = padW(fusion_W[:D])
    bp = jnp.zeros((8, DP), f32)
    bp = bp.at[0, :D].set(agg_b[0]).at[1, :D].set(agg_b[1])
    bp = bp.at[2, :D].set(agg_b[2]).at[3, :D].set(aggL_b)
    def pad_gates(w):
        out = jnp.zeros((DP, 4 * DP), f32)
        for g in range(4):
            out = out.at[:D, g * DP:g * DP + D].set(w[:, g * D:(g + 1) * D])
        return out
    Wih = pad_gates(W_ih)
    Whh = pad_gates(W_hh)
    xb = jnp.zeros((1, 4 * DP), f32)
    bsum = b_ih + b_hh
    for g in range(4):
        xb = xb.at[0, g * DP:g * DP + D].set(bsum[g * D:(g + 1) * D])
    v0 = emb_r[0] @ fusion_W[D:] + fusion_b
    v1 = emb_r[1] @ fusion_W[D:] + fusion_b
    rt = response.T.reshape(-1)[:N_REAL]
    er2 = jnp.where((rt[:, None] > 0), jnp.pad(v1, (0, DP - D)),
                    jnp.pad(v0, (0, DP - D)))
    er2 = jnp.pad(er2, ((0, NT - N_REAL), (0, 0)))
    wv = jnp.stack([jnp.full((DP,), w1_q, f32), jnp.full((DP,), w2_q, f32)])
    qvec = jnp.pad(q_W @ w_W[:D, 0], (0, DP - D))[None, :]
    kvec = jnp.pad(k_W @ w_W[D:, 0], (0, DP - D))[None, :]
    P1 = (jnp.kron(jnp.eye(TB, dtype=f32), jnp.ones((1, 4), f32)) * 0.25)
    P2 = (jnp.kron(jnp.eye(4 * TB, dtype=f32), jnp.ones((1, 4), f32)) * 0.25)

    xproj, u_norm = _k1(E0, E1, E2, E3, F0, F1, F2, F3, QN, SK, er2,
                        W0, W1, W2, WL, Fw1, Wih, P1, P2, bp, xb, wv, qvec)

    hist_t = _k2(xproj.reshape(G1, B, 4 * DP), Whh)

    Gq_t = E0[:T * B].reshape(T, B, DP)
    gqb = jnp.pad(Gq_t.transpose(1, 0, 2), ((0, 0), (0, TP - T), (0, 0)))
    gsb = jnp.pad(Gq_t[1:].transpose(1, 0, 2),
                  ((0, 0), (0, TP - Tm1), (0, 0)))
    histb_p = jnp.pad(hist_t.transpose(1, 0, 2), ((0, 0), (0, TP - Tm1), (0, 0)))
    unb = jnp.pad(u_norm[:N_REAL].reshape(Tm1, B, DP).transpose(1, 0, 2),
                  ((0, 0), (0, TP - Tm1), (0, 0)))

    P = _k3(gqb, gsb, histb_p, unb, kvec)
    p = jax.nn.sigmoid(P[:, :Tm1, 0])
    return jnp.concatenate([jnp.zeros((B, 1), f32), p], axis=1)


# speedup vs baseline: 6.7467x; 1.1083x over previous
import functools

import jax
import jax.numpy as jnp
from jax import lax
from jax.experimental import pallas as pl
from jax.experimental.pallas import tpu as pltpu
from jax.experimental.pallas import tpu_sc as plsc

NQ, NS, NU = 10000, 1000, 20000
D, B, T = 100, 64, 50
NB = 4
RANK_K = 10
MAX_S = 4
DP = 128
Tm1 = T - 1
N_REAL = Tm1 * B
NWK = 32
TW = 104
NT = NWK * TW
TP = 56
NEG = float(-3.0e38)


def _sc_gather_fn(qflat, uflat, qn_tbl, sn_tbl, un_tbl, qn2_tbl, qsk_tbl,
                  embq, embs, embu, embq2,
                  E0, E1, E2, E3, F0, F1, F2, F3, QN, SK,
                  qts, uts, qnx,
                  x4a, n1_f, x4b, n2_f, x4c, n3_f, sk_f, R,
                  s0, s1, s2, s3):
    sems = (s0, s1, s2, s3)
    wid = lax.axis_index("s") * 2 + lax.axis_index("c")
    base = wid * TW

    def take16(v, idx):
        dn = lax.GatherDimensionNumbers(offset_dims=(),
                                        collapsed_slice_dims=(0,),
                                        start_index_map=(0,))
        return lax.gather(v, idx[:, None], dn, slice_sizes=(1,),
                          mode=lax.GatherScatterMode.PROMISE_IN_BOUNDS)

    def expand4(src1d, dst1d, nchunks2):
        def body(c, carry):
            it = lax.iota(jnp.int32, 16)
            sub = lax.shift_right_logical(it, 2)
            cl = lax.bitwise_and(it, 3)
            v = src1d[pl.ds(c * 8, 16)]
            a = take16(v, sub)
            bvals = take16(v, sub + 4)
            dst1d[pl.ds(c * 32, 16)] = a * 4 + cl
            dst1d[pl.ds(c * 32 + 16, 16)] = bvals * 4 + cl
            return carry
        lax.fori_loop(0, nchunks2, body, 0)

    def expand4cm(src1d, dst1d, nchunks, seg):
        def body(c, carry):
            v = src1d[pl.ds(c * 16, 16)]
            for m in range(4):
                dst1d[pl.ds(m * seg + c * 16, 16)] = v * 4 + m
            return carry
        lax.fori_loop(0, nchunks, body, 0)

    def elem_level(flat_tbl, idx_ref, nparts, dst1d):
        for g0 in range(0, nparts, 4):
            gcnt = min(4, nparts - g0)
            hs = []
            for j in range(gcnt):
                p = g0 + j
                hs.append(pltpu.async_copy(
                    flat_tbl.at[idx_ref.at[pl.ds(p * TW, TW)]],
                    dst1d.at[pl.ds(p * TW, TW)], sems[j]))
            for h in hs:
                h.wait()

    def row_level(emb_tbl, idx_ref, nparts, out_hbm, out_base, idx_base=0):
        for g0 in range(0, nparts, 4):
            gcnt = min(4, nparts - g0)
            hs = []
            for j in range(gcnt):
                p = g0 + j
                hs.append(pltpu.async_copy(
                    emb_tbl.at[idx_ref.at[pl.ds(idx_base + p * TW, TW)]],
                    R.at[pl.ds(j * TW, TW)], sems[j]))
            for h in hs:
                h.wait()
            pltpu.sync_copy(R.at[pl.ds(0, gcnt * TW)],
                            out_hbm.at[pl.ds(out_base + g0 * TW, gcnt * TW)])

    def tree(idx0_ref, hop1_flat, hop2_flat, emb_even, emb_odd,
             O0, O1, O2, O3):
        row_level(emb_even, idx0_ref, 1, O0, base)
        expand4(idx0_ref, x4a, 13)
        elem_level(hop1_flat, x4a, 4, n1_f)
        row_level(emb_odd, n1_f, 4, O1, 4 * base)
        expand4(n1_f, x4b, 52)
        elem_level(hop2_flat, x4b, 16, n2_f)
        row_level(emb_even, n2_f, 16, O2, 16 * base)
        expand4cm(n2_f, x4c, 104, 16 * TW)
        elem_level(hop1_flat, x4c, 64, n3_f)
        for m in range(4):
            row_level(emb_odd, n3_f, 16, O3,
                      m * (NT * 16) + 16 * base, idx_base=m * 16 * TW)

    pltpu.sync_copy(qflat.at[pl.ds(base, TW + 8)], qts)
    pltpu.sync_copy(uflat.at[pl.ds(base, TW + 8)], uts)
    pltpu.sync_copy(qflat.at[pl.ds(base + B, TW + 8)], qnx)

    tree(qts, qn_tbl, sn_tbl, embq, embs, E0, E1, E2, E3)
    tree(uts, un_tbl, qn2_tbl, embu, embq2, F0, F1, F2, F3)

    row_level(embq, qnx, 1, QN, base)
    expand4(qnx, x4a, 13)
    elem_level(qsk_tbl, x4a, 4, sk_f)
    row_level(embs, sk_f, 4, SK, 4 * base)


def _sc_gather(qflat, uflat, qn_tbl, sn_tbl, un_tbl, qn2_tbl, qsk_tbl,
               embq, embs, embu, embq2):
    f32, i32 = jnp.float32, jnp.int32
    out_type = [
        jax.ShapeDtypeStruct((NT, DP), f32),
        jax.ShapeDtypeStruct((NT * 4, DP), f32),
        jax.ShapeDtypeStruct((NT * 16, DP), f32),
        jax.ShapeDtypeStruct((NT * 64, DP), f32),
        jax.ShapeDtypeStruct((NT, DP), f32),
        jax.ShapeDtypeStruct((NT * 4, DP), f32),
        jax.ShapeDtypeStruct((NT * 16, DP), f32),
        jax.ShapeDtypeStruct((NT * 64, DP), f32),
        jax.ShapeDtypeStruct((NT, DP), f32),
        jax.ShapeDtypeStruct((NT * 4, DP), f32),
    ]
    scratch = [
        pltpu.VMEM((TW + 8,), i32), pltpu.VMEM((TW + 8,), i32),
        pltpu.VMEM((TW + 8,), i32),
        pltpu.VMEM((4 * TW,), i32), pltpu.VMEM((4 * TW + 32,), i32),
        pltpu.VMEM((16 * TW,), i32), pltpu.VMEM((16 * TW + 32,), i32),
        pltpu.VMEM((64 * TW,), i32), pltpu.VMEM((64 * TW,), i32),
        pltpu.VMEM((4 * TW,), i32),
        pltpu.VMEM((4 * TW, DP), f32),
        pltpu.SemaphoreType.DMA, pltpu.SemaphoreType.DMA,
        pltpu.SemaphoreType.DMA, pltpu.SemaphoreType.DMA,
    ]
    mesh = plsc.VectorSubcoreMesh(core_axis_name="c", subcore_axis_name="s")
    return pl.kernel(_sc_gather_fn, mesh=mesh, out_type=out_type,
                     scratch_types=scratch)(
        qflat, uflat, qn_tbl, sn_tbl, un_tbl, qn2_tbl, qsk_tbl,
        embq, embs, embu, embq2)


TB = 64
G1 = NT // TB


def _k1_fn(e0, e1f, e2f, e3a, e3b, e3c, e3d,
           f0, f1f, f2f, f3a, f3b, f3c, f3d,
           qn, skg, er2,
           W0, W1, W2, WL, Fw1, Wih, P1, P2, bp, xb, wv, qvec,
           xp_out, un_out):
    r = jax.nn.relu

    def dot(a, b):
        return jnp.dot(a, b, preferred_element_type=jnp.float32)

    b0 = bp[0:1, :]
    b1 = bp[1:2, :]
    b2 = bp[2:3, :]
    bL = bp[3:4, :]

    def tree(x0, x1f, x2f, x3a, x3b, x3c, x3d):
        m3 = (x3a[...] + x3b[...] + x3c[...] + x3d[...]) * 0.25
        A2 = r(dot(m3 + x2f[...], W2[...]) + b2)
        A1 = r(dot(dot(P2[...], x2f[...]) + x1f[...], W1[...]) + b1)
        A0 = r(dot(dot(P1[...], x1f[...]) + x0[...], W0[...]) + b0)
        B0 = r(dot(dot(P1[...], A1) + A0, W0[...]) + b0)
        B1 = r(dot(dot(P2[...], A2) + A1, W1[...]) + b1)
        C0 = r(dot(dot(P1[...], B1) + B0, W0[...]) + b0)
        return r(dot(C0, WL[...]) + bL)

    g1 = tree(e0, e1f, e2f, e3a, e3b, e3c, e3d)
    g2 = tree(f0, f1f, f2f, f3a, f3b, f3c, f3d)
    ehat = g1 * wv[0:1, :] + g2 * wv[1:2, :]
    e_t = r(dot(ehat, Fw1[...]) + er2[...])
    xp_out[...] = dot(e_t, Wih[...]) + xb[...]

    qnv = qn[...]
    skv = skg[...]
    qv = qvec[...]
    qd0 = jnp.sum(qnv * qv, axis=-1, keepdims=True)
    qds = jnp.sum(skv * qv[None], axis=-1)
    qall = jnp.concatenate([qd0, qds], axis=1)
    mq = jnp.max(qall, axis=1, keepdims=True)
    wq = jnp.exp(qall - mq)
    u = wq[:, 0:1] * qnv
    for j in range(MAX_S):
        u = u + wq[:, j + 1:j + 2] * skv[:, j, :]
    sq = jnp.sum(wq, axis=1, keepdims=True)
    un_out[...] = u / sq


def _k1(e0, e1, e2, e3, f0, f1, f2, f3, qn, sk, er2,
        W0, W1, W2, WL, Fw1, Wih, P1, P2, bp, xb, wv, qvec):
    f32 = jnp.float32
    full = lambda shape: pl.BlockSpec(shape, lambda i: tuple(0 for _ in shape))

    def bank(m):
        return pl.BlockSpec((16 * TB, DP), lambda i, m=m: (m * G1 + i, 0))

    specs = [
        pl.BlockSpec((TB, DP), lambda i: (i, 0)),
        pl.BlockSpec((4 * TB, DP), lambda i: (i, 0)),
        pl.BlockSpec((16 * TB, DP), lambda i: (i, 0)),
        bank(0), bank(1), bank(2), bank(3),
        pl.BlockSpec((TB, DP), lambda i: (i, 0)),
        pl.BlockSpec((4 * TB, DP), lambda i: (i, 0)),
        pl.BlockSpec((16 * TB, DP), lambda i: (i, 0)),
        bank(0), bank(1), bank(2), bank(3),
        pl.BlockSpec((TB, DP), lambda i: (i, 0)),
        pl.BlockSpec((TB, 4, DP), lambda i: (i, 0, 0)),
        pl.BlockSpec((TB, DP), lambda i: (i, 0)),
        full((DP, DP)), full((DP, DP)), full((DP, DP)), full((DP, DP)),
        full((DP, DP)), full((DP, 4 * DP)),
        full((TB, 4 * TB)), full((4 * TB, 16 * TB)),
        full((8, DP)), full((1, 4 * DP)), full((2, DP)), full((1, DP)),
    ]
    return pl.pallas_call(
        _k1_fn,
        grid=(G1,),
        in_specs=specs,
        out_specs=[pl.BlockSpec((TB, 4 * DP), lambda i: (i, 0)),
                   pl.BlockSpec((TB, DP), lambda i: (i, 0))],
        out_shape=[jax.ShapeDtypeStruct((NT, 4 * DP), f32),
                   jax.ShapeDtypeStruct((NT, DP), f32)],
    )(e0, e1, e2, e3, e3, e3, e3, f0, f1, f2, f3, f3, f3, f3,
      qn, sk.reshape(NT, 4, DP), er2,
      W0, W1, W2, WL, Fw1, Wih, P1, P2, bp, xb, wv, qvec)


def _k2_fn(xp_ref, whh, hist_out, h, c):
    t = pl.program_id(0)

    @pl.when(t == 0)
    def _():
        h[...] = jnp.zeros((B, DP), jnp.float32)
        c[...] = jnp.zeros((B, DP), jnp.float32)

    g = xp_ref[0] + jnp.dot(h[...], whh[...],
                            preferred_element_type=jnp.float32)
    i_g = g[:, 0:DP]
    f_g = g[:, DP:2 * DP]
    g_g = g[:, 2 * DP:3 * DP]
    o_g = g[:, 3 * DP:4 * DP]
    c2 = jax.nn.sigmoid(f_g) * c[...] + jax.nn.sigmoid(i_g) * jnp.tanh(g_g)
    h2 = jax.nn.sigmoid(o_g) * jnp.tanh(c2)
    h[...] = h2
    c[...] = c2
    hist_out[0] = h2


def _k2(xproj, whh):
    return pl.pallas_call(
        _k2_fn,
        grid=(Tm1,),
        in_specs=[pl.BlockSpec((1, B, 4 * DP), lambda t: (t, 0, 0)),
                  pl.BlockSpec((DP, 4 * DP), lambda t: (0, 0))],
        out_specs=pl.BlockSpec((1, B, DP), lambda t: (t, 0, 0)),
        out_shape=jax.ShapeDtypeStruct((Tm1, B, DP), jnp.float32),
        scratch_shapes=[pltpu.VMEM((B, DP), jnp.float32),
                        pltpu.VMEM((B, DP), jnp.float32)],
    )(xproj, whh)


def _k3_fn(gq_ref, gs_ref, h_ref, u_ref, kv_ref, out_ref):
    Gv = gq_ref[0]
    Gs = gs_ref[0]
    H = h_ref[0]
    U = u_ref[0]
    dims = (((1,), (1,)), ((), ()))
    Srow = lax.dot_general(Gs, Gv, dims,
                           preferred_element_type=jnp.float32)
    tt = lax.broadcasted_iota(jnp.int32, (TP, TP), 0)
    ta = lax.broadcasted_iota(jnp.int32, (TP, TP), 1)
    valid = (ta < tt) & (tt < Tm1)
    Sm = jnp.where(valid, Srow, NEG)
    gtr = jnp.sum((Sm[:, :, None] > Sm[:, None, :]).astype(jnp.int32),
                  axis=1)
    i1 = lax.broadcasted_iota(jnp.int32, (TP, TP, TP), 1)
    i2 = lax.broadcasted_iota(jnp.int32, (TP, TP, TP), 2)
    eqc = jnp.sum(((Sm[:, :, None] == Sm[:, None, :]) & (i1 < i2))
                  .astype(jnp.int32), axis=1)
    rank = gtr + eqc
    sel = (valid & (rank < RANK_K)) | (valid & (tt < RANK_K)) \
        | ((ta == tt) & (tt < Tm1))
    hk = lax.dot_general(kv_ref[...], H, dims,
                         preferred_element_type=jnp.float32)
    hkb = jnp.broadcast_to(hk, (TP, TP))
    mk = jnp.max(jnp.where(sel, hkb, NEG), axis=1, keepdims=True)
    wk = jnp.where(sel, jnp.exp(hkb - mk), 0.0)
    sk = jnp.sum(wk, axis=1, keepdims=True)
    V = jnp.dot(wk, H, preferred_element_type=jnp.float32)
    num = jnp.sum(U * V, axis=1, keepdims=True)
    p = num / sk
    out_ref[0] = jnp.broadcast_to(p, (TP, DP))


def _k3(gqb, gsb, histb, unb, kvec):
    spec = pl.BlockSpec((1, TP, DP), lambda b: (b, 0, 0))
    return pl.pallas_call(
        _k3_fn,
        grid=(B,),
        in_specs=[spec, spec, spec, spec,
                  pl.BlockSpec((1, DP), lambda b: (0, 0))],
        out_specs=spec,
        out_shape=jax.ShapeDtypeStruct((B, TP, DP), jnp.float32),
    )(gqb, gsb, histb, unb, kvec)


def kernel(emb_q, emb_q2, emb_s, emb_u, emb_r, w1_q, w2_q, W_ih, W_hh,
           b_ih, b_hh, fusion_W, fusion_b, agg_W, agg_b, aggL_W, aggL_b,
           q_W, q_b, k_W, k_b, w_W, w_b, user, question, response, mask,
           q_neighbors, s_neighbors, u_neighbors, q_neighbors_2,
           q_skill_idx):
    f32 = jnp.float32
    padD = lambda x: jnp.pad(x, ((0, 0), (0, DP - D)))
    embq_p = padD(emb_q)
    embs_p = padD(emb_s)
    embu_p = padD(emb_u)
    embq2_p = padD(emb_q2)

    qflat = jnp.pad(question.T.reshape(-1).astype(jnp.int32),
                    (0, 3520 - T * B))
    uflat = jnp.pad(user.T.reshape(-1).astype(jnp.int32), (0, 3520 - T * B))

    outs = _sc_gather(qflat, uflat,
                      q_neighbors.astype(jnp.int32).reshape(-1),
                      s_neighbors.astype(jnp.int32).reshape(-1),
                      u_neighbors.astype(jnp.int32).reshape(-1),
                      q_neighbors_2.astype(jnp.int32).reshape(-1),
                      q_skill_idx.astype(jnp.int32).reshape(-1),
                      embq_p, embs_p, embu_p, embq2_p)
    E0, E1, E2, E3, F0, F1, F2, F3, QN, SK = outs

    padW = lambda w: jnp.pad(w, ((0, DP - w.shape[0]), (0, DP - w.shape[1])))
    W0, W1, W2 = padW(agg_W[0]), padW(agg_W[1]), padW(agg_W[2])
    WL = padW(aggL_W)
    Fw1 = padW(fusion_W[:D])
    bp = jnp.zeros((8, DP), f32)
    bp = bp.at[0, :D].set(agg_b[0]).at[1, :D].set(agg_b[1])
    bp = bp.at[2, :D].set(agg_b[2]).at[3, :D].set(aggL_b)
    def pad_gates(w):
        out = jnp.zeros((DP, 4 * DP), f32)
        for g in range(4):
            out = out.at[:D, g * DP:g * DP + D].set(w[:, g * D:(g + 1) * D])
        return out
    Wih = pad_gates(W_ih)
    Whh = pad_gates(W_hh)
    xb = jnp.zeros((1, 4 * DP), f32)
    bsum = b_ih + b_hh
    for g in range(4):
        xb = xb.at[0, g * DP:g * DP + D].set(bsum[g * D:(g + 1) * D])
    v0 = emb_r[0] @ fusion_W[D:] + fusion_b
    v1 = emb_r[1] @ fusion_W[D:] + fusion_b
    rt = response.T.reshape(-1)[:N_REAL]
    er2 = jnp.where((rt[:, None] > 0), jnp.pad(v1, (0, DP - D)),
                    jnp.pad(v0, (0, DP - D)))
    er2 = jnp.pad(er2, ((0, NT - N_REAL), (0, 0)))
    wv = jnp.stack([jnp.full((DP,), w1_q, f32), jnp.full((DP,), w2_q, f32)])
    qvec = jnp.pad(q_W @ w_W[:D, 0], (0, DP - D))[None, :]
    kvec = jnp.pad(k_W @ w_W[D:, 0], (0, DP - D))[None, :]
    P1 = (jnp.kron(jnp.eye(TB, dtype=f32), jnp.ones((1, 4), f32)) * 0.25)
    P2 = (jnp.kron(jnp.eye(4 * TB, dtype=f32), jnp.ones((1, 4), f32)) * 0.25)

    xproj, u_norm = _k1(E0, E1, E2, E3, F0, F1, F2, F3, QN, SK, er2,
                        W0, W1, W2, WL, Fw1, Wih, P1, P2, bp, xb, wv, qvec)

    hist_t = _k2(xproj.reshape(G1, B, 4 * DP), Whh)

    Gq_t = E0[:T * B].reshape(T, B, DP)
    gqb = jnp.pad(Gq_t.transpose(1, 0, 2), ((0, 0), (0, TP - T), (0, 0)))
    gsb = jnp.pad(Gq_t[1:].transpose(1, 0, 2),
                  ((0, 0), (0, TP - Tm1), (0, 0)))
    histb_p = jnp.pad(hist_t.transpose(1, 0, 2), ((0, 0), (0, TP - Tm1), (0, 0)))
    unb = jnp.pad(u_norm[:N_REAL].reshape(Tm1, B, DP).transpose(1, 0, 2),
                  ((0, 0), (0, TP - Tm1), (0, 0)))

    P = _k3(gqb, gsb, histb_p, unb, kvec)
    p = jax.nn.sigmoid(P[:, :Tm1, 0])
    return jnp.concatenate([jnp.zeros((B, 1), f32), p], axis=1)


# speedup vs baseline: 6.8513x; 1.0155x over previous
import functools

import jax
import jax.numpy as jnp
from jax import lax
from jax.experimental import pallas as pl
from jax.experimental.pallas import tpu as pltpu
from jax.experimental.pallas import tpu_sc as plsc

NQ, NS, NU = 10000, 1000, 20000
D, B, T = 100, 64, 50
NB = 4
RANK_K = 10
MAX_S = 4
DP = 128
Tm1 = T - 1
N_REAL = Tm1 * B
NWK = 32
TW = 104
NT = NWK * TW
TP = 56
NEG = float(-3.0e38)


def _sc_gather_fn(qflat, uflat, qn_tbl, sn_tbl, un_tbl, qn2_tbl, qsk_tbl,
                  embq, embs, embu, embq2,
                  E0, E1, E2, E3, F0, F1, F2, F3, QN, SK,
                  qts, uts, qnx,
                  x4a, n1_f, x4b, n2_f, x4c, n3_f, sk_f, R,
                  s0, s1, s2, s3, s4, s5, s6, s7):
    sems = (s0, s1, s2, s3, s4, s5, s6, s7)
    wid = lax.axis_index("s") * 2 + lax.axis_index("c")
    base = wid * TW

    def take16(v, idx):
        dn = lax.GatherDimensionNumbers(offset_dims=(),
                                        collapsed_slice_dims=(0,),
                                        start_index_map=(0,))
        return lax.gather(v, idx[:, None], dn, slice_sizes=(1,),
                          mode=lax.GatherScatterMode.PROMISE_IN_BOUNDS)

    def expand4(src1d, dst1d, nchunks2):
        def body(c, carry):
            it = lax.iota(jnp.int32, 16)
            sub = lax.shift_right_logical(it, 2)
            cl = lax.bitwise_and(it, 3)
            v = src1d[pl.ds(c * 8, 16)]
            a = take16(v, sub)
            bvals = take16(v, sub + 4)
            dst1d[pl.ds(c * 32, 16)] = a * 4 + cl
            dst1d[pl.ds(c * 32 + 16, 16)] = bvals * 4 + cl
            return carry
        lax.fori_loop(0, nchunks2, body, 0)

    def expand4cm(src1d, dst1d, nchunks, seg):
        def body(c, carry):
            v = src1d[pl.ds(c * 16, 16)]
            for m in range(4):
                dst1d[pl.ds(m * seg + c * 16, 16)] = v * 4 + m
            return carry
        lax.fori_loop(0, nchunks, body, 0)

    def elem_level(flat_tbl, idx_ref, nparts, dst1d):
        for g0 in range(0, nparts, 8):
            gcnt = min(8, nparts - g0)
            hs = []
            for j in range(gcnt):
                p = g0 + j
                hs.append(pltpu.async_copy(
                    flat_tbl.at[idx_ref.at[pl.ds(p * TW, TW)]],
                    dst1d.at[pl.ds(p * TW, TW)], sems[j]))
            for h in hs:
                h.wait()

    def row_level(emb_tbl, idx_ref, nparts, out_hbm, out_base, idx_base=0):
        for g0 in range(0, nparts, 8):
            gcnt = min(8, nparts - g0)
            hs = []
            for j in range(gcnt):
                p = g0 + j
                hs.append(pltpu.async_copy(
                    emb_tbl.at[idx_ref.at[pl.ds(idx_base + p * TW, TW)]],
                    R.at[pl.ds(j * TW, TW)], sems[j]))
            for h in hs:
                h.wait()
            pltpu.sync_copy(R.at[pl.ds(0, gcnt * TW)],
                            out_hbm.at[pl.ds(out_base + g0 * TW, gcnt * TW)])

    def tree(idx0_ref, hop1_flat, hop2_flat, emb_even, emb_odd,
             O0, O1, O2, O3):
        row_level(emb_even, idx0_ref, 1, O0, base)
        expand4(idx0_ref, x4a, 13)
        elem_level(hop1_flat, x4a, 4, n1_f)
        row_level(emb_odd, n1_f, 4, O1, 4 * base)
        expand4(n1_f, x4b, 52)
        elem_level(hop2_flat, x4b, 16, n2_f)
        row_level(emb_even, n2_f, 16, O2, 16 * base)
        expand4cm(n2_f, x4c, 104, 16 * TW)
        elem_level(hop1_flat, x4c, 64, n3_f)
        for m in range(4):
            row_level(emb_odd, n3_f, 16, O3,
                      m * (NT * 16) + 16 * base, idx_base=m * 16 * TW)

    pltpu.sync_copy(qflat.at[pl.ds(base, TW + 8)], qts)
    pltpu.sync_copy(uflat.at[pl.ds(base, TW + 8)], uts)
    pltpu.sync_copy(qflat.at[pl.ds(base + B, TW + 8)], qnx)

    tree(qts, qn_tbl, sn_tbl, embq, embs, E0, E1, E2, E3)
    tree(uts, un_tbl, qn2_tbl, embu, embq2, F0, F1, F2, F3)

    row_level(embq, qnx, 1, QN, base)
    expand4(qnx, x4a, 13)
    elem_level(qsk_tbl, x4a, 4, sk_f)
    row_level(embs, sk_f, 4, SK, 4 * base)


def _sc_gather(qflat, uflat, qn_tbl, sn_tbl, un_tbl, qn2_tbl, qsk_tbl,
               embq, embs, embu, embq2):
    f32, i32 = jnp.float32, jnp.int32
    out_type = [
        jax.ShapeDtypeStruct((NT, DP), f32),
        jax.ShapeDtypeStruct((NT * 4, DP), f32),
        jax.ShapeDtypeStruct((NT * 16, DP), f32),
        jax.ShapeDtypeStruct((NT * 64, DP), f32),
        jax.ShapeDtypeStruct((NT, DP), f32),
        jax.ShapeDtypeStruct((NT * 4, DP), f32),
        jax.ShapeDtypeStruct((NT * 16, DP), f32),
        jax.ShapeDtypeStruct((NT * 64, DP), f32),
        jax.ShapeDtypeStruct((NT, DP), f32),
        jax.ShapeDtypeStruct((NT * 4, DP), f32),
    ]
    scratch = [
        pltpu.VMEM((TW + 8,), i32), pltpu.VMEM((TW + 8,), i32),
        pltpu.VMEM((TW + 8,), i32),
        pltpu.VMEM((4 * TW,), i32), pltpu.VMEM((4 * TW + 32,), i32),
        pltpu.VMEM((16 * TW,), i32), pltpu.VMEM((16 * TW + 32,), i32),
        pltpu.VMEM((64 * TW,), i32), pltpu.VMEM((64 * TW,), i32),
        pltpu.VMEM((4 * TW,), i32),
        pltpu.VMEM((8 * TW, DP), f32),
    ] + [pltpu.SemaphoreType.DMA] * 8
    mesh = plsc.VectorSubcoreMesh(core_axis_name="c", subcore_axis_name="s")
    return pl.kernel(_sc_gather_fn, mesh=mesh, out_type=out_type,
                     scratch_types=scratch)(
        qflat, uflat, qn_tbl, sn_tbl, un_tbl, qn2_tbl, qsk_tbl,
        embq, embs, embu, embq2)


TB = 64
G1 = NT // TB


def _k1_fn(e0, e1f, e2f, e3a, e3b, e3c, e3d,
           f0, f1f, f2f, f3a, f3b, f3c, f3d,
           qn, skg, er2,
           W0, W1, W2, WL, Fw1, Wih, P1, P2, bp, xb, wv, qvec,
           xp_out, un_out):
    r = jax.nn.relu

    def dot(a, b):
        return jnp.dot(a, b, preferred_element_type=jnp.float32)

    b0 = bp[0:1, :]
    b1 = bp[1:2, :]
    b2 = bp[2:3, :]
    bL = bp[3:4, :]

    def tree(x0, x1f, x2f, x3a, x3b, x3c, x3d):
        m3 = (x3a[...] + x3b[...] + x3c[...] + x3d[...]) * 0.25
        A2 = r(dot(m3 + x2f[...], W2[...]) + b2)
        A1 = r(dot(dot(P2[...], x2f[...]) + x1f[...], W1[...]) + b1)
        A0 = r(dot(dot(P1[...], x1f[...]) + x0[...], W0[...]) + b0)
        B0 = r(dot(dot(P1[...], A1) + A0, W0[...]) + b0)
        B1 = r(dot(dot(P2[...], A2) + A1, W1[...]) + b1)
        C0 = r(dot(dot(P1[...], B1) + B0, W0[...]) + b0)
        return r(dot(C0, WL[...]) + bL)

    g1 = tree(e0, e1f, e2f, e3a, e3b, e3c, e3d)
    g2 = tree(f0, f1f, f2f, f3a, f3b, f3c, f3d)
    ehat = g1 * wv[0:1, :] + g2 * wv[1:2, :]
    e_t = r(dot(ehat, Fw1[...]) + er2[...])
    xp_out[...] = dot(e_t, Wih[...]) + xb[...]

    qnv = qn[...]
    skv = skg[...]
    qv = qvec[...]
    qd0 = jnp.sum(qnv * qv, axis=-1, keepdims=True)
    qds = jnp.sum(skv * qv[None], axis=-1)
    qall = jnp.concatenate([qd0, qds], axis=1)
    mq = jnp.max(qall, axis=1, keepdims=True)
    wq = jnp.exp(qall - mq)
    u = wq[:, 0:1] * qnv
    for j in range(MAX_S):
        u = u + wq[:, j + 1:j + 2] * skv[:, j, :]
    sq = jnp.sum(wq, axis=1, keepdims=True)
    un_out[...] = u / sq


def _k1(e0, e1, e2, e3, f0, f1, f2, f3, qn, sk, er2,
        W0, W1, W2, WL, Fw1, Wih, P1, P2, bp, xb, wv, qvec):
    f32 = jnp.float32
    full = lambda shape: pl.BlockSpec(shape, lambda i: tuple(0 for _ in shape))

    def bank(m):
        return pl.BlockSpec((16 * TB, DP), lambda i, m=m: (m * G1 + i, 0))

    specs = [
        pl.BlockSpec((TB, DP), lambda i: (i, 0)),
        pl.BlockSpec((4 * TB, DP), lambda i: (i, 0)),
        pl.BlockSpec((16 * TB, DP), lambda i: (i, 0)),
        bank(0), bank(1), bank(2), bank(3),
        pl.BlockSpec((TB, DP), lambda i: (i, 0)),
        pl.BlockSpec((4 * TB, DP), lambda i: (i, 0)),
        pl.BlockSpec((16 * TB, DP), lambda i: (i, 0)),
        bank(0), bank(1), bank(2), bank(3),
        pl.BlockSpec((TB, DP), lambda i: (i, 0)),
        pl.BlockSpec((TB, 4, DP), lambda i: (i, 0, 0)),
        pl.BlockSpec((TB, DP), lambda i: (i, 0)),
        full((DP, DP)), full((DP, DP)), full((DP, DP)), full((DP, DP)),
        full((DP, DP)), full((DP, 4 * DP)),
        full((TB, 4 * TB)), full((4 * TB, 16 * TB)),
        full((8, DP)), full((1, 4 * DP)), full((2, DP)), full((1, DP)),
    ]
    return pl.pallas_call(
        _k1_fn,
        grid=(G1,),
        in_specs=specs,
        out_specs=[pl.BlockSpec((TB, 4 * DP), lambda i: (i, 0)),
                   pl.BlockSpec((TB, DP), lambda i: (i, 0))],
        out_shape=[jax.ShapeDtypeStruct((NT, 4 * DP), f32),
                   jax.ShapeDtypeStruct((NT, DP), f32)],
    )(e0, e1, e2, e3, e3, e3, e3, f0, f1, f2, f3, f3, f3, f3,
      qn, sk.reshape(NT, 4, DP), er2,
      W0, W1, W2, WL, Fw1, Wih, P1, P2, bp, xb, wv, qvec)


def _k2_fn(xp_ref, whh, hist_out, h, c):
    t = pl.program_id(0)

    @pl.when(t == 0)
    def _():
        h[...] = jnp.zeros((B, DP), jnp.float32)
        c[...] = jnp.zeros((B, DP), jnp.float32)

    g = xp_ref[0] + jnp.dot(h[...], whh[...],
                            preferred_element_type=jnp.float32)
    i_g = g[:, 0:DP]
    f_g = g[:, DP:2 * DP]
    g_g = g[:, 2 * DP:3 * DP]
    o_g = g[:, 3 * DP:4 * DP]
    c2 = jax.nn.sigmoid(f_g) * c[...] + jax.nn.sigmoid(i_g) * jnp.tanh(g_g)
    h2 = jax.nn.sigmoid(o_g) * jnp.tanh(c2)
    h[...] = h2
    c[...] = c2
    hist_out[0] = h2


def _k2(xproj, whh):
    return pl.pallas_call(
        _k2_fn,
        grid=(Tm1,),
        in_specs=[pl.BlockSpec((1, B, 4 * DP), lambda t: (t, 0, 0)),
                  pl.BlockSpec((DP, 4 * DP), lambda t: (0, 0))],
        out_specs=pl.BlockSpec((1, B, DP), lambda t: (t, 0, 0)),
        out_shape=jax.ShapeDtypeStruct((Tm1, B, DP), jnp.float32),
        scratch_shapes=[pltpu.VMEM((B, DP), jnp.float32),
                        pltpu.VMEM((B, DP), jnp.float32)],
    )(xproj, whh)


def _k3_fn(gq_ref, gs_ref, h_ref, u_ref, kv_ref, out_ref):
    Gv = gq_ref[0]
    Gs = gs_ref[0]
    H = h_ref[0]
    U = u_ref[0]
    dims = (((1,), (1,)), ((), ()))
    Srow = lax.dot_general(Gs, Gv, dims,
                           preferred_element_type=jnp.float32)
    tt = lax.broadcasted_iota(jnp.int32, (TP, TP), 0)
    ta = lax.broadcasted_iota(jnp.int32, (TP, TP), 1)
    valid = (ta < tt) & (tt < Tm1)
    Sm = jnp.where(valid, Srow, NEG)
    gtr = jnp.sum((Sm[:, :, None] > Sm[:, None, :]).astype(jnp.int32),
                  axis=1)
    i1 = lax.broadcasted_iota(jnp.int32, (TP, TP, TP), 1)
    i2 = lax.broadcasted_iota(jnp.int32, (TP, TP, TP), 2)
    eqc = jnp.sum(((Sm[:, :, None] == Sm[:, None, :]) & (i1 < i2))
                  .astype(jnp.int32), axis=1)
    rank = gtr + eqc
    sel = (valid & (rank < RANK_K)) | (valid & (tt < RANK_K)) \
        | ((ta == tt) & (tt < Tm1))
    hk = lax.dot_general(kv_ref[...], H, dims,
                         preferred_element_type=jnp.float32)
    hkb = jnp.broadcast_to(hk, (TP, TP))
    mk = jnp.max(jnp.where(sel, hkb, NEG), axis=1, keepdims=True)
    wk = jnp.where(sel, jnp.exp(hkb - mk), 0.0)
    sk = jnp.sum(wk, axis=1, keepdims=True)
    V = jnp.dot(wk, H, preferred_element_type=jnp.float32)
    num = jnp.sum(U * V, axis=1, keepdims=True)
    p = num / sk
    out_ref[0] = jnp.broadcast_to(p, (TP, DP))


def _k3(gqb, gsb, histb, unb, kvec):
    spec = pl.BlockSpec((1, TP, DP), lambda b: (b, 0, 0))
    return pl.pallas_call(
        _k3_fn,
        grid=(B,),
        in_specs=[spec, spec, spec, spec,
                  pl.BlockSpec((1, DP), lambda b: (0, 0))],
        out_specs=spec,
        out_shape=jax.ShapeDtypeStruct((B, TP, DP), jnp.float32),
    )(gqb, gsb, histb, unb, kvec)


def kernel(emb_q, emb_q2, emb_s, emb_u, emb_r, w1_q, w2_q, W_ih, W_hh,
           b_ih, b_hh, fusion_W, fusion_b, agg_W, agg_b, aggL_W, aggL_b,
           q_W, q_b, k_W, k_b, w_W, w_b, user, question, response, mask,
           q_neighbors, s_neighbors, u_neighbors, q_neighbors_2,
           q_skill_idx):
    f32 = jnp.float32
    padD = lambda x: jnp.pad(x, ((0, 0), (0, DP - D)))
    embq_p = padD(emb_q)
    embs_p = padD(emb_s)
    embu_p = padD(emb_u)
    embq2_p = padD(emb_q2)

    qflat = jnp.pad(question.T.reshape(-1).astype(jnp.int32),
                    (0, 3520 - T * B))
    uflat = jnp.pad(user.T.reshape(-1).astype(jnp.int32), (0, 3520 - T * B))

    outs = _sc_gather(qflat, uflat,
                      q_neighbors.astype(jnp.int32).reshape(-1),
                      s_neighbors.astype(jnp.int32).reshape(-1),
                      u_neighbors.astype(jnp.int32).reshape(-1),
                      q_neighbors_2.astype(jnp.int32).reshape(-1),
                      q_skill_idx.astype(jnp.int32).reshape(-1),
                      embq_p, embs_p, embu_p, embq2_p)
    E0, E1, E2, E3, F0, F1, F2, F3, QN, SK = outs

    padW = lambda w: jnp.pad(w, ((0, DP - w.shape[0]), (0, DP - w.shape[1])))
    W0, W1, W2 = padW(agg_W[0]), padW(agg_W[1]), padW(agg_W[2])
    WL = padW(aggL_W)
    Fw1 = padW(fusion_W[:D])
    bp = jnp.zeros((8, DP), f32)
    bp = bp.at[0, :D].set(agg_b[0]).at[1, :D].set(agg_b[1])
    bp = bp.at[2, :D].set(agg_b[2]).at[3, :D].set(aggL_b)
    def pad_gates(w):
        out = jnp.zeros((DP, 4 * DP), f32)
        for g in range(4):
            out = out.at[:D, g * DP:g * DP + D].set(w[:, g * D:(g + 1) * D])
        return out
    Wih = pad_gates(W_ih)
    Whh = pad_gates(W_hh)
    xb = jnp.zeros((1, 4 * DP), f32)
    bsum = b_ih + b_hh
    for g in range(4):
        xb = xb.at[0, g * DP:g * DP + D].set(bsum[g * D:(g + 1) * D])
    v0 = emb_r[0] @ fusion_W[D:] + fusion_b
    v1 = emb_r[1] @ fusion_W[D:] + fusion_b
    rt = response.T.reshape(-1)[:N_REAL]
    er2 = jnp.where((rt[:, None] > 0), jnp.pad(v1, (0, DP - D)),
                    jnp.pad(v0, (0, DP - D)))
    er2 = jnp.pad(er2, ((0, NT - N_REAL), (0, 0)))
    wv = jnp.stack([jnp.full((DP,), w1_q, f32), jnp.full((DP,), w2_q, f32)])
    qvec = jnp.pad(q_W @ w_W[:D, 0], (0, DP - D))[None, :]
    kvec = jnp.pad(k_W @ w_W[D:, 0], (0, DP - D))[None, :]
    P1 = (jnp.kron(jnp.eye(TB, dtype=f32), jnp.ones((1, 4), f32)) * 0.25)
    P2 = (jnp.kron(jnp.eye(4 * TB, dtype=f32), jnp.ones((1, 4), f32)) * 0.25)

    xproj, u_norm = _k1(E0, E1, E2, E3, F0, F1, F2, F3, QN, SK, er2,
                        W0, W1, W2, WL, Fw1, Wih, P1, P2, bp, xb, wv, qvec)

    hist_t = _k2(xproj.reshape(G1, B, 4 * DP), Whh)

    Gq_t = E0[:T * B].reshape(T, B, DP)
    gqb = jnp.pad(Gq_t.transpose(1, 0, 2), ((0, 0), (0, TP - T), (0, 0)))
    gsb = jnp.pad(Gq_t[1:].transpose(1, 0, 2),
                  ((0, 0), (0, TP - Tm1), (0, 0)))
    histb_p = jnp.pad(hist_t.transpose(1, 0, 2), ((0, 0), (0, TP - Tm1), (0, 0)))
    unb = jnp.pad(u_norm[:N_REAL].reshape(Tm1, B, DP).transpose(1, 0, 2),
                  ((0, 0), (0, TP - Tm1), (0, 0)))

    P = _k3(gqb, gsb, histb_p, unb, kvec)
    p = jax.nn.sigmoid(P[:, :Tm1, 0])
    return jnp.concatenate([jnp.zeros((B, 1), f32), p], axis=1)


# speedup vs baseline: 6.8879x; 1.0053x over previous
import functools

import jax
import jax.numpy as jnp
from jax import lax
from jax.experimental import pallas as pl
from jax.experimental.pallas import tpu as pltpu
from jax.experimental.pallas import tpu_sc as plsc

NQ, NS, NU = 10000, 1000, 20000
D, B, T = 100, 64, 50
NB = 4
RANK_K = 10
MAX_S = 4
DP = 128
Tm1 = T - 1
N_REAL = Tm1 * B
NWK = 32
TW = 104
NT = NWK * TW
TP = 56
NEG = float(-3.0e38)


def _sc_gather_fn(qflat, uflat, qn_tbl, sn_tbl, un_tbl, qn2_tbl, qsk_tbl,
                  embq, embs, embu, embq2,
                  E0, E1, E2, E3, F0, F1, F2, F3, QN, SK,
                  qts, uts, qnx,
                  x4a, n1_f, x4b, n2_f, x4c, n3_f, sk_f, R,
                  s0, s1, s2, s3, s4, s5, s6, s7):
    sems = (s0, s1, s2, s3, s4, s5, s6, s7)
    wid = lax.axis_index("s") * 2 + lax.axis_index("c")
    base = wid * TW

    def take16(v, idx):
        dn = lax.GatherDimensionNumbers(offset_dims=(),
                                        collapsed_slice_dims=(0,),
                                        start_index_map=(0,))
        return lax.gather(v, idx[:, None], dn, slice_sizes=(1,),
                          mode=lax.GatherScatterMode.PROMISE_IN_BOUNDS)

    def expand4(src1d, dst1d, nchunks2):
        def body(c, carry):
            it = lax.iota(jnp.int32, 16)
            sub = lax.shift_right_logical(it, 2)
            cl = lax.bitwise_and(it, 3)
            v = src1d[pl.ds(c * 8, 16)]
            a = take16(v, sub)
            bvals = take16(v, sub + 4)
            dst1d[pl.ds(c * 32, 16)] = a * 4 + cl
            dst1d[pl.ds(c * 32 + 16, 16)] = bvals * 4 + cl
            return carry
        lax.fori_loop(0, nchunks2, body, 0)

    def expand4cm(src1d, dst1d, nchunks, seg):
        def body(c, carry):
            v = src1d[pl.ds(c * 16, 16)]
            for m in range(4):
                dst1d[pl.ds(m * seg + c * 16, 16)] = v * 4 + m
            return carry
        lax.fori_loop(0, nchunks, body, 0)

    def elem_level(flat_tbl, idx_ref, nparts, dst1d):
        for g0 in range(0, nparts, 8):
            gcnt = min(8, nparts - g0)
            pltpu.async_copy(
                flat_tbl.at[idx_ref.at[pl.ds(g0 * TW, gcnt * TW)]],
                dst1d.at[pl.ds(g0 * TW, gcnt * TW)], sems[0]).wait()

    def row_level(emb_tbl, idx_ref, nparts, out_hbm, out_base, idx_base=0):
        for g0 in range(0, nparts, 8):
            gcnt = min(8, nparts - g0)
            pltpu.async_copy(
                emb_tbl.at[idx_ref.at[pl.ds(idx_base + g0 * TW, gcnt * TW)]],
                R.at[pl.ds(0, gcnt * TW)], sems[0]).wait()
            pltpu.sync_copy(R.at[pl.ds(0, gcnt * TW)],
                            out_hbm.at[pl.ds(out_base + g0 * TW, gcnt * TW)])

    def tree(idx0_ref, hop1_flat, hop2_flat, emb_even, emb_odd,
             O0, O1, O2, O3):
        row_level(emb_even, idx0_ref, 1, O0, base)
        expand4(idx0_ref, x4a, 13)
        elem_level(hop1_flat, x4a, 4, n1_f)
        row_level(emb_odd, n1_f, 4, O1, 4 * base)
        expand4(n1_f, x4b, 52)
        elem_level(hop2_flat, x4b, 16, n2_f)
        row_level(emb_even, n2_f, 16, O2, 16 * base)
        expand4cm(n2_f, x4c, 104, 16 * TW)
        elem_level(hop1_flat, x4c, 64, n3_f)
        for m in range(4):
            row_level(emb_odd, n3_f, 16, O3,
                      m * (NT * 16) + 16 * base, idx_base=m * 16 * TW)

    pltpu.sync_copy(qflat.at[pl.ds(base, TW + 8)], qts)
    pltpu.sync_copy(uflat.at[pl.ds(base, TW + 8)], uts)
    pltpu.sync_copy(qflat.at[pl.ds(base + B, TW + 8)], qnx)

    tree(qts, qn_tbl, sn_tbl, embq, embs, E0, E1, E2, E3)
    tree(uts, un_tbl, qn2_tbl, embu, embq2, F0, F1, F2, F3)

    row_level(embq, qnx, 1, QN, base)
    expand4(qnx, x4a, 13)
    elem_level(qsk_tbl, x4a, 4, sk_f)
    row_level(embs, sk_f, 4, SK, 4 * base)


def _sc_gather(qflat, uflat, qn_tbl, sn_tbl, un_tbl, qn2_tbl, qsk_tbl,
               embq, embs, embu, embq2):
    f32, i32 = jnp.float32, jnp.int32
    out_type = [
        jax.ShapeDtypeStruct((NT, DP), f32),
        jax.ShapeDtypeStruct((NT * 4, DP), f32),
        jax.ShapeDtypeStruct((NT * 16, DP), f32),
        jax.ShapeDtypeStruct((NT * 64, DP), f32),
        jax.ShapeDtypeStruct((NT, DP), f32),
        jax.ShapeDtypeStruct((NT * 4, DP), f32),
        jax.ShapeDtypeStruct((NT * 16, DP), f32),
        jax.ShapeDtypeStruct((NT * 64, DP), f32),
        jax.ShapeDtypeStruct((NT, DP), f32),
        jax.ShapeDtypeStruct((NT * 4, DP), f32),
    ]
    scratch = [
        pltpu.VMEM((TW + 8,), i32), pltpu.VMEM((TW + 8,), i32),
        pltpu.VMEM((TW + 8,), i32),
        pltpu.VMEM((4 * TW,), i32), pltpu.VMEM((4 * TW + 32,), i32),
        pltpu.VMEM((16 * TW,), i32), pltpu.VMEM((16 * TW + 32,), i32),
        pltpu.VMEM((64 * TW,), i32), pltpu.VMEM((64 * TW,), i32),
        pltpu.VMEM((4 * TW,), i32),
        pltpu.VMEM((8 * TW, DP), f32),
    ] + [pltpu.SemaphoreType.DMA] * 8
    mesh = plsc.VectorSubcoreMesh(core_axis_name="c", subcore_axis_name="s")
    return pl.kernel(_sc_gather_fn, mesh=mesh, out_type=out_type,
                     scratch_types=scratch)(
        qflat, uflat, qn_tbl, sn_tbl, un_tbl, qn2_tbl, qsk_tbl,
        embq, embs, embu, embq2)


TB = 64
G1 = NT // TB


def _k1_fn(e0, e1f, e2f, e3a, e3b, e3c, e3d,
           f0, f1f, f2f, f3a, f3b, f3c, f3d,
           qn, skg, er2,
           W0, W1, W2, WL, Fw1, Wih, P1, P2, bp, xb, wv, qvec,
           xp_out, un_out):
    r = jax.nn.relu

    def dot(a, b):
        return jnp.dot(a, b, preferred_element_type=jnp.float32)

    b0 = bp[0:1, :]
    b1 = bp[1:2, :]
    b2 = bp[2:3, :]
    bL = bp[3:4, :]

    def tree(x0, x1f, x2f, x3a, x3b, x3c, x3d):
        m3 = (x3a[...] + x3b[...] + x3c[...] + x3d[...]) * 0.25
        A2 = r(dot(m3 + x2f[...], W2[...]) + b2)
        A1 = r(dot(dot(P2[...], x2f[...]) + x1f[...], W1[...]) + b1)
        A0 = r(dot(dot(P1[...], x1f[...]) + x0[...], W0[...]) + b0)
        B0 = r(dot(dot(P1[...], A1) + A0, W0[...]) + b0)
        B1 = r(dot(dot(P2[...], A2) + A1, W1[...]) + b1)
        C0 = r(dot(dot(P1[...], B1) + B0, W0[...]) + b0)
        return r(dot(C0, WL[...]) + bL)

    g1 = tree(e0, e1f, e2f, e3a, e3b, e3c, e3d)
    g2 = tree(f0, f1f, f2f, f3a, f3b, f3c, f3d)
    ehat = g1 * wv[0:1, :] + g2 * wv[1:2, :]
    e_t = r(dot(ehat, Fw1[...]) + er2[...])
    xp_out[...] = dot(e_t, Wih[...]) + xb[...]

    qnv = qn[...]
    skv = skg[...]
    qv = qvec[...]
    qd0 = jnp.sum(qnv * qv, axis=-1, keepdims=True)
    qds = jnp.sum(skv * qv[None], axis=-1)
    qall = jnp.concatenate([qd0, qds], axis=1)
    mq = jnp.max(qall, axis=1, keepdims=True)
    wq = jnp.exp(qall - mq)
    u = wq[:, 0:1] * qnv
    for j in range(MAX_S):
        u = u + wq[:, j + 1:j + 2] * skv[:, j, :]
    sq = jnp.sum(wq, axis=1, keepdims=True)
    un_out[...] = u / sq


def _k1(e0, e1, e2, e3, f0, f1, f2, f3, qn, sk, er2,
        W0, W1, W2, WL, Fw1, Wih, P1, P2, bp, xb, wv, qvec):
    f32 = jnp.float32
    full = lambda shape: pl.BlockSpec(shape, lambda i: tuple(0 for _ in shape))

    def bank(m):
        return pl.BlockSpec((16 * TB, DP), lambda i, m=m: (m * G1 + i, 0))

    specs = [
        pl.BlockSpec((TB, DP), lambda i: (i, 0)),
        pl.BlockSpec((4 * TB, DP), lambda i: (i, 0)),
        pl.BlockSpec((16 * TB, DP), lambda i: (i, 0)),
        bank(0), bank(1), bank(2), bank(3),
        pl.BlockSpec((TB, DP), lambda i: (i, 0)),
        pl.BlockSpec((4 * TB, DP), lambda i: (i, 0)),
        pl.BlockSpec((16 * TB, DP), lambda i: (i, 0)),
        bank(0), bank(1), bank(2), bank(3),
        pl.BlockSpec((TB, DP), lambda i: (i, 0)),
        pl.BlockSpec((TB, 4, DP), lambda i: (i, 0, 0)),
        pl.BlockSpec((TB, DP), lambda i: (i, 0)),
        full((DP, DP)), full((DP, DP)), full((DP, DP)), full((DP, DP)),
        full((DP, DP)), full((DP, 4 * DP)),
        full((TB, 4 * TB)), full((4 * TB, 16 * TB)),
        full((8, DP)), full((1, 4 * DP)), full((2, DP)), full((1, DP)),
    ]
    return pl.pallas_call(
        _k1_fn,
        grid=(G1,),
        in_specs=specs,
        out_specs=[pl.BlockSpec((TB, 4 * DP), lambda i: (i, 0)),
                   pl.BlockSpec((TB, DP), lambda i: (i, 0))],
        out_shape=[jax.ShapeDtypeStruct((NT, 4 * DP), f32),
                   jax.ShapeDtypeStruct((NT, DP), f32)],
    )(e0, e1, e2, e3, e3, e3, e3, f0, f1, f2, f3, f3, f3, f3,
      qn, sk.reshape(NT, 4, DP), er2,
      W0, W1, W2, WL, Fw1, Wih, P1, P2, bp, xb, wv, qvec)


def _k2_fn(xp_ref, whh, hist_out, h, c):
    t = pl.program_id(0)

    @pl.when(t == 0)
    def _():
        h[...] = jnp.zeros((B, DP), jnp.float32)
        c[...] = jnp.zeros((B, DP), jnp.float32)

    g = xp_ref[0] + jnp.dot(h[...], whh[...],
                            preferred_element_type=jnp.float32)
    i_g = g[:, 0:DP]
    f_g = g[:, DP:2 * DP]
    g_g = g[:, 2 * DP:3 * DP]
    o_g = g[:, 3 * DP:4 * DP]
    c2 = jax.nn.sigmoid(f_g) * c[...] + jax.nn.sigmoid(i_g) * jnp.tanh(g_g)
    h2 = jax.nn.sigmoid(o_g) * jnp.tanh(c2)
    h[...] = h2
    c[...] = c2
    hist_out[0] = h2


def _k2(xproj, whh):
    return pl.pallas_call(
        _k2_fn,
        grid=(Tm1,),
        in_specs=[pl.BlockSpec((1, B, 4 * DP), lambda t: (t, 0, 0)),
                  pl.BlockSpec((DP, 4 * DP), lambda t: (0, 0))],
        out_specs=pl.BlockSpec((1, B, DP), lambda t: (t, 0, 0)),
        out_shape=jax.ShapeDtypeStruct((Tm1, B, DP), jnp.float32),
        scratch_shapes=[pltpu.VMEM((B, DP), jnp.float32),
                        pltpu.VMEM((B, DP), jnp.float32)],
    )(xproj, whh)


def _k3_fn(gq_ref, gs_ref, h_ref, u_ref, kv_ref, out_ref):
    Gv = gq_ref[0]
    Gs = gs_ref[0]
    H = h_ref[0]
    U = u_ref[0]
    dims = (((1,), (1,)), ((), ()))
    Srow = lax.dot_general(Gs, Gv, dims,
                           preferred_element_type=jnp.float32)
    tt = lax.broadcasted_iota(jnp.int32, (TP, TP), 0)
    ta = lax.broadcasted_iota(jnp.int32, (TP, TP), 1)
    valid = (ta < tt) & (tt < Tm1)
    Sm = jnp.where(valid, Srow, NEG)
    gtr = jnp.sum((Sm[:, :, None] > Sm[:, None, :]).astype(jnp.int32),
                  axis=1)
    i1 = lax.broadcasted_iota(jnp.int32, (TP, TP, TP), 1)
    i2 = lax.broadcasted_iota(jnp.int32, (TP, TP, TP), 2)
    eqc = jnp.sum(((Sm[:, :, None] == Sm[:, None, :]) & (i1 < i2))
                  .astype(jnp.int32), axis=1)
    rank = gtr + eqc
    sel = (valid & (rank < RANK_K)) | (valid & (tt < RANK_K)) \
        | ((ta == tt) & (tt < Tm1))
    hk = lax.dot_general(kv_ref[...], H, dims,
                         preferred_element_type=jnp.float32)
    hkb = jnp.broadcast_to(hk, (TP, TP))
    mk = jnp.max(jnp.where(sel, hkb, NEG), axis=1, keepdims=True)
    wk = jnp.where(sel, jnp.exp(hkb - mk), 0.0)
    sk = jnp.sum(wk, axis=1, keepdims=True)
    V = jnp.dot(wk, H, preferred_element_type=jnp.float32)
    num = jnp.sum(U * V, axis=1, keepdims=True)
    p = num / sk
    out_ref[0] = jnp.broadcast_to(p, (TP, DP))


def _k3(gqb, gsb, histb, unb, kvec):
    spec = pl.BlockSpec((1, TP, DP), lambda b: (b, 0, 0))
    return pl.pallas_call(
        _k3_fn,
        grid=(B,),
        in_specs=[spec, spec, spec, spec,
                  pl.BlockSpec((1, DP), lambda b: (0, 0))],
        out_specs=spec,
        out_shape=jax.ShapeDtypeStruct((B, TP, DP), jnp.float32),
    )(gqb, gsb, histb, unb, kvec)


def kernel(emb_q, emb_q2, emb_s, emb_u, emb_r, w1_q, w2_q, W_ih, W_hh,
           b_ih, b_hh, fusion_W, fusion_b, agg_W, agg_b, aggL_W, aggL_b,
           q_W, q_b, k_W, k_b, w_W, w_b, user, question, response, mask,
           q_neighbors, s_neighbors, u_neighbors, q_neighbors_2,
           q_skill_idx):
    f32 = jnp.float32
    padD = lambda x: jnp.pad(x, ((0, 0), (0, DP - D)))
    embq_p = padD(emb_q)
    embs_p = padD(emb_s)
    embu_p = padD(emb_u)
    embq2_p = padD(emb_q2)

    qflat = jnp.pad(question.T.reshape(-1).astype(jnp.int32),
                    (0, 3520 - T * B))
    uflat = jnp.pad(user.T.reshape(-1).astype(jnp.int32), (0, 3520 - T * B))

    outs = _sc_gather(qflat, uflat,
                      q_neighbors.astype(jnp.int32).reshape(-1),
                      s_neighbors.astype(jnp.int32).reshape(-1),
                      u_neighbors.astype(jnp.int32).reshape(-1),
                      q_neighbors_2.astype(jnp.int32).reshape(-1),
                      q_skill_idx.astype(jnp.int32).reshape(-1),
                      embq_p, embs_p, embu_p, embq2_p)
    E0, E1, E2, E3, F0, F1, F2, F3, QN, SK = outs

    padW = lambda w: jnp.pad(w, ((0, DP - w.shape[0]), (0, DP - w.shape[1])))
    W0, W1, W2 = padW(agg_W[0]), padW(agg_W[1]), padW(agg_W[2])
    WL = padW(aggL_W)
    Fw1 = padW(fusion_W[:D])
    bp = jnp.zeros((8, DP), f32)
    bp = bp.at[0, :D].set(agg_b[0]).at[1, :D].set(agg_b[1])
    bp = bp.at[2, :D].set(agg_b[2]).at[3, :D].set(aggL_b)
    def pad_gates(w):
        out = jnp.zeros((DP, 4 * DP), f32)
        for g in range(4):
            out = out.at[:D, g * DP:g * DP + D].set(w[:, g * D:(g + 1) * D])
        return out
    Wih = pad_gates(W_ih)
    Whh = pad_gates(W_hh)
    xb = jnp.zeros((1, 4 * DP), f32)
    bsum = b_ih + b_hh
    for g in range(4):
        xb = xb.at[0, g * DP:g * DP + D].set(bsum[g * D:(g + 1) * D])
    v0 = emb_r[0] @ fusion_W[D:] + fusion_b
    v1 = emb_r[1] @ fusion_W[D:] + fusion_b
    rt = response.T.reshape(-1)[:N_REAL]
    er2 = jnp.where((rt[:, None] > 0), jnp.pad(v1, (0, DP - D)),
                    jnp.pad(v0, (0, DP - D)))
    er2 = jnp.pad(er2, ((0, NT - N_REAL), (0, 0)))
    wv = jnp.stack([jnp.full((DP,), w1_q, f32), jnp.full((DP,), w2_q, f32)])
    qvec = jnp.pad(q_W @ w_W[:D, 0], (0, DP - D))[None, :]
    kvec = jnp.pad(k_W @ w_W[D:, 0], (0, DP - D))[None, :]
    P1 = (jnp.kron(jnp.eye(TB, dtype=f32), jnp.ones((1, 4), f32)) * 0.25)
    P2 = (jnp.kron(jnp.eye(4 * TB, dtype=f32), jnp.ones((1, 4), f32)) * 0.25)

    xproj, u_norm = _k1(E0, E1, E2, E3, F0, F1, F2, F3, QN, SK, er2,
                        W0, W1, W2, WL, Fw1, Wih, P1, P2, bp, xb, wv, qvec)

    hist_t = _k2(xproj.reshape(G1, B, 4 * DP), Whh)

    Gq_t = E0[:T * B].reshape(T, B, DP)
    gqb = jnp.pad(Gq_t.transpose(1, 0, 2), ((0, 0), (0, TP - T), (0, 0)))
    gsb = jnp.pad(Gq_t[1:].transpose(1, 0, 2),
                  ((0, 0), (0, TP - Tm1), (0, 0)))
    histb_p = jnp.pad(hist_t.transpose(1, 0, 2), ((0, 0), (0, TP - Tm1), (0, 0)))
    unb = jnp.pad(u_norm[:N_REAL].reshape(Tm1, B, DP).transpose(1, 0, 2),
                  ((0, 0), (0, TP - Tm1), (0, 0)))

    P = _k3(gqb, gsb, histb_p, unb, kvec)
    p = jax.nn.sigmoid(P[:, :Tm1, 0])
    return jnp.concatenate([jnp.zeros((B, 1), f32), p], axis=1)


# speedup vs baseline: 6.9878x; 1.0145x over previous
import functools

import jax
import jax.numpy as jnp
from jax import lax
from jax.experimental import pallas as pl
from jax.experimental.pallas import tpu as pltpu
from jax.experimental.pallas import tpu_sc as plsc

NQ, NS, NU = 10000, 1000, 20000
D, B, T = 100, 64, 50
NB = 4
RANK_K = 10
MAX_S = 4
DP = 128
Tm1 = T - 1
N_REAL = Tm1 * B
NWK = 32
TW = 104
NT = NWK * TW
TP = 56
NEG = float(-3.0e38)


def _sc_gather_fn(qflat, uflat, qn_tbl, sn_tbl, un_tbl, qn2_tbl, qsk_tbl,
                  embq, embs, embu, embq2,
                  E0, E1, N2, F0, F1, M2, QN, SK, E3Q, E3U,
                  qts, uts, qnx,
                  x4a, n1_f, x4b, n2_f, x4c, n3_f, sk_f, R,
                  s0, s1, s2, s3, s4, s5, s6, s7):
    sems = (s0, s1, s2, s3, s4, s5, s6, s7)
    wid = lax.axis_index("s") * 2 + lax.axis_index("c")
    base = wid * TW

    def take16(v, idx):
        dn = lax.GatherDimensionNumbers(offset_dims=(),
                                        collapsed_slice_dims=(0,),
                                        start_index_map=(0,))
        return lax.gather(v, idx[:, None], dn, slice_sizes=(1,),
                          mode=lax.GatherScatterMode.PROMISE_IN_BOUNDS)

    def expand4(src1d, dst1d, nchunks2):
        def body(c, carry):
            it = lax.iota(jnp.int32, 16)
            sub = lax.shift_right_logical(it, 2)
            cl = lax.bitwise_and(it, 3)
            v = src1d[pl.ds(c * 8, 16)]
            a = take16(v, sub)
            bvals = take16(v, sub + 4)
            dst1d[pl.ds(c * 32, 16)] = a * 4 + cl
            dst1d[pl.ds(c * 32 + 16, 16)] = bvals * 4 + cl
            return carry
        lax.fori_loop(0, nchunks2, body, 0)

    def expand4cm(src1d, dst1d, nchunks, seg):
        def body(c, carry):
            v = src1d[pl.ds(c * 16, 16)]
            for m in range(4):
                dst1d[pl.ds(m * seg + c * 16, 16)] = v * 4 + m
            return carry
        lax.fori_loop(0, nchunks, body, 0)

    def elem_level(flat_tbl, idx_ref, nparts, dst1d):
        for g0 in range(0, nparts, 8):
            gcnt = min(8, nparts - g0)
            pltpu.async_copy(
                flat_tbl.at[idx_ref.at[pl.ds(g0 * TW, gcnt * TW)]],
                dst1d.at[pl.ds(g0 * TW, gcnt * TW)], sems[0]).wait()

    def row_level(emb_tbl, idx_ref, nparts, out_hbm, out_base, idx_base=0):
        for g0 in range(0, nparts, 8):
            gcnt = min(8, nparts - g0)
            pltpu.async_copy(
                emb_tbl.at[idx_ref.at[pl.ds(idx_base + g0 * TW, gcnt * TW)]],
                R.at[pl.ds(0, gcnt * TW)], sems[0]).wait()
            pltpu.sync_copy(R.at[pl.ds(0, gcnt * TW)],
                            out_hbm.at[pl.ds(out_base + g0 * TW, gcnt * TW)])

    def extract4(src1d, nvals, dst1d, bstride):
        it = lax.iota(jnp.int32, 16)
        qsel1 = it < 4
        qsel2 = it < 8
        qsel3 = it < 12
        def body(c, carry):
            vs = [src1d[pl.ds(c * 64 + k * 16, 16)] for k in range(4)]
            for m in range(4):
                idxv = (it & 3) * 4 + m
                sh = [take16(v, idxv) for v in vs]
                out = jnp.where(qsel1, sh[0],
                                jnp.where(qsel2, sh[1],
                                          jnp.where(qsel3, sh[2], sh[3])))
                dst1d[pl.ds(m * bstride + c * 16, 16)] = out
            return carry
        lax.fori_loop(0, nvals // 16, body, 0)

    def tree(idx0_ref, hop1_flat, hop2_flat, emb_even, emb_odd,
             O0, O1, O2f):
        row_level(emb_even, idx0_ref, 1, O0, base)
        expand4(idx0_ref, x4a, 13)
        elem_level(hop1_flat, x4a, 4, n1_f)
        row_level(emb_odd, n1_f, 4, O1, 4 * base)
        expand4(n1_f, x4b, 52)
        elem_level(hop2_flat, x4b, 16, n2_f)
        pltpu.sync_copy(n2_f.at[pl.ds(0, 16 * TW)],
                        O2f.at[pl.ds(16 * base, 16 * TW)])

    def m3_build(nb_flat, emb_tbl, nper, OB, nbank_rows):
        pltpu.sync_copy(nb_flat.at[pl.ds(wid * 4 * nper, 4 * nper)],
                        x4c.at[pl.ds(0, 4 * nper)])
        extract4(x4c, nper, n3_f, 1664)
        for m in range(4):
            pltpu.async_copy(
                emb_tbl.at[n3_f.at[pl.ds(m * 1664, nper)]],
                R.at[pl.ds(0, nper)], sems[0]).wait()
            pltpu.sync_copy(R.at[pl.ds(0, nper)],
                            OB.at[pl.ds(m * nbank_rows + wid * nper, nper)])

    pltpu.sync_copy(qflat.at[pl.ds(base, TW + 8)], qts)
    pltpu.sync_copy(uflat.at[pl.ds(base, TW + 8)], uts)
    pltpu.sync_copy(qflat.at[pl.ds(base + B, TW + 8)], qnx)

    tree(qts, qn_tbl, sn_tbl, embq, embs, E0, E1, N2)
    tree(uts, un_tbl, qn2_tbl, embu, embq2, F0, F1, M2)
    m3_build(qn_tbl, embs, 320, E3Q, 10240)
    m3_build(un_tbl, embq2, 640, E3U, 20480)

    row_level(embq, qnx, 1, QN, base)
    expand4(qnx, x4a, 13)
    elem_level(qsk_tbl, x4a, 4, sk_f)
    row_level(embs, sk_f, 4, SK, 4 * base)


def _sc_gather(qflat, uflat, qn_tbl, sn_tbl, un_tbl, qn2_tbl, qsk_tbl,
               embq, embs, embu, embq2):
    f32, i32 = jnp.float32, jnp.int32
    out_type = [
        jax.ShapeDtypeStruct((NT, DP), f32),
        jax.ShapeDtypeStruct((NT * 4, DP), f32),
        jax.ShapeDtypeStruct((NT * 16,), i32),
        jax.ShapeDtypeStruct((NT, DP), f32),
        jax.ShapeDtypeStruct((NT * 4, DP), f32),
        jax.ShapeDtypeStruct((NT * 16,), i32),
        jax.ShapeDtypeStruct((NT, DP), f32),
        jax.ShapeDtypeStruct((NT * 4, DP), f32),
        jax.ShapeDtypeStruct((4 * 10240, DP), f32),
        jax.ShapeDtypeStruct((4 * 20480, DP), f32),
    ]
    scratch = [
        pltpu.VMEM((TW + 8,), i32), pltpu.VMEM((TW + 8,), i32),
        pltpu.VMEM((TW + 8,), i32),
        pltpu.VMEM((4 * TW,), i32), pltpu.VMEM((4 * TW + 32,), i32),
        pltpu.VMEM((16 * TW,), i32), pltpu.VMEM((16 * TW + 32,), i32),
        pltpu.VMEM((64 * TW,), i32), pltpu.VMEM((64 * TW,), i32),
        pltpu.VMEM((4 * TW,), i32),
        pltpu.VMEM((8 * TW, DP), f32),
    ] + [pltpu.SemaphoreType.DMA] * 8
    mesh = plsc.VectorSubcoreMesh(core_axis_name="c", subcore_axis_name="s")
    return pl.kernel(_sc_gather_fn, mesh=mesh, out_type=out_type,
                     scratch_types=scratch)(
        qflat, uflat, qn_tbl, sn_tbl, un_tbl, qn2_tbl, qsk_tbl,
        embq, embs, embu, embq2)


def _km_fn(emb, b0, b1, b2, b3, out):
    out[:, 0:DP] = emb[...]
    out[:, DP:2 * DP] = (b0[...] + b1[...] + b2[...] + b3[...]) * 0.25


def _km(emb_pad, banks, nrows):
    gm = nrows // 1024

    def bankspec(m):
        return pl.BlockSpec((1024, DP), lambda i, m=m: (m * gm + i, 0))

    return pl.pallas_call(
        _km_fn,
        grid=(gm,),
        in_specs=[pl.BlockSpec((1024, DP), lambda i: (i, 0)),
                  bankspec(0), bankspec(1), bankspec(2), bankspec(3)],
        out_specs=pl.BlockSpec((1024, 2 * DP), lambda i: (i, 0)),
        out_shape=jax.ShapeDtypeStruct((nrows, 2 * DP), jnp.float32),
    )(emb_pad, banks, banks, banks, banks)


def _sc2_fn(TQ, TU, N2, M2, E2M, F2M, idxv, R2, s0):
    wid = lax.axis_index("s") * 2 + lax.axis_index("c")
    base16 = wid * 16 * TW
    for idx_hbm, tbl, out in ((N2, TQ, E2M), (M2, TU, F2M)):
        pltpu.sync_copy(idx_hbm.at[pl.ds(base16, 16 * TW)], idxv)
        for p in range(4):
            pltpu.async_copy(tbl.at[idxv.at[pl.ds(p * 416, 416)]],
                             R2, s0).wait()
            pltpu.sync_copy(R2, out.at[pl.ds(base16 + p * 416, 416)])


def _sc2(TQ, TU, N2, M2):
    f32, i32 = jnp.float32, jnp.int32
    out_type = [jax.ShapeDtypeStruct((NT * 16, 2 * DP), f32),
                jax.ShapeDtypeStruct((NT * 16, 2 * DP), f32)]
    scratch = [pltpu.VMEM((16 * TW,), i32),
               pltpu.VMEM((416, 2 * DP), f32),
               pltpu.SemaphoreType.DMA]
    mesh = plsc.VectorSubcoreMesh(core_axis_name="c", subcore_axis_name="s")
    return pl.kernel(_sc2_fn, mesh=mesh, out_type=out_type,
                     scratch_types=scratch)(TQ, TU, N2, M2)


TB = 64
G1 = NT // TB


def _k1_fn(e0, e1f, em, f0, f1f, fm,
           qn, skg, er2,
           W0, W1, W2, WL, Fw1, Wih, P1, P2, bp, xb, wv, qvec,
           xp_out, un_out):
    r = jax.nn.relu

    def dot(a, b):
        return jnp.dot(a, b, preferred_element_type=jnp.float32)

    b0 = bp[0:1, :]
    b1 = bp[1:2, :]
    b2 = bp[2:3, :]
    bL = bp[3:4, :]

    def tree(x0, x1f, xm):
        x2f = xm[:, 0:DP]
        m3 = xm[:, DP:2 * DP]
        A2 = r(dot(m3 + x2f, W2[...]) + b2)
        A1 = r(dot(dot(P2[...], x2f) + x1f[...], W1[...]) + b1)
        A0 = r(dot(dot(P1[...], x1f[...]) + x0[...], W0[...]) + b0)
        B0 = r(dot(dot(P1[...], A1) + A0, W0[...]) + b0)
        B1 = r(dot(dot(P2[...], A2) + A1, W1[...]) + b1)
        C0 = r(dot(dot(P1[...], B1) + B0, W0[...]) + b0)
        return r(dot(C0, WL[...]) + bL)

    g1 = tree(e0, e1f[...], em[...])
    g2 = tree(f0, f1f, fm[...])
    ehat = g1 * wv[0:1, :] + g2 * wv[1:2, :]
    e_t = r(dot(ehat, Fw1[...]) + er2[...])
    xp_out[...] = dot(e_t, Wih[...]) + xb[...]

    qnv = qn[...]
    skv = skg[...]
    qv = qvec[...]
    qd0 = jnp.sum(qnv * qv, axis=-1, keepdims=True)
    qds = jnp.sum(skv * qv[None], axis=-1)
    qall = jnp.concatenate([qd0, qds], axis=1)
    mq = jnp.max(qall, axis=1, keepdims=True)
    wq = jnp.exp(qall - mq)
    u = wq[:, 0:1] * qnv
    for j in range(MAX_S):
        u = u + wq[:, j + 1:j + 2] * skv[:, j, :]
    sq = jnp.sum(wq, axis=1, keepdims=True)
    un_out[...] = u / sq


def _k1(e0, e1, em, f0, f1, fm, qn, sk, er2,
        W0, W1, W2, WL, Fw1, Wih, P1, P2, bp, xb, wv, qvec):
    f32 = jnp.float32
    full = lambda shape: pl.BlockSpec(shape, lambda i: tuple(0 for _ in shape))

    specs = [
        pl.BlockSpec((TB, DP), lambda i: (i, 0)),
        pl.BlockSpec((4 * TB, DP), lambda i: (i, 0)),
        pl.BlockSpec((16 * TB, 2 * DP), lambda i: (i, 0)),
        pl.BlockSpec((TB, DP), lambda i: (i, 0)),
        pl.BlockSpec((4 * TB, DP), lambda i: (i, 0)),
        pl.BlockSpec((16 * TB, 2 * DP), lambda i: (i, 0)),
        pl.BlockSpec((TB, DP), lambda i: (i, 0)),
        pl.BlockSpec((TB, 4, DP), lambda i: (i, 0, 0)),
        pl.BlockSpec((TB, DP), lambda i: (i, 0)),
        full((DP, DP)), full((DP, DP)), full((DP, DP)), full((DP, DP)),
        full((DP, DP)), full((DP, 4 * DP)),
        full((TB, 4 * TB)), full((4 * TB, 16 * TB)),
        full((8, DP)), full((1, 4 * DP)), full((2, DP)), full((1, DP)),
    ]
    return pl.pallas_call(
        _k1_fn,
        grid=(G1,),
        in_specs=specs,
        out_specs=[pl.BlockSpec((TB, 4 * DP), lambda i: (i, 0)),
                   pl.BlockSpec((TB, DP), lambda i: (i, 0))],
        out_shape=[jax.ShapeDtypeStruct((NT, 4 * DP), f32),
                   jax.ShapeDtypeStruct((NT, DP), f32)],
    )(e0, e1, em, f0, f1, fm,
      qn, sk.reshape(NT, 4, DP), er2,
      W0, W1, W2, WL, Fw1, Wih, P1, P2, bp, xb, wv, qvec)


def _k2_fn(xp_ref, whh, hist_out, h, c):
    t = pl.program_id(0)

    @pl.when(t == 0)
    def _():
        h[...] = jnp.zeros((B, DP), jnp.float32)
        c[...] = jnp.zeros((B, DP), jnp.float32)

    g = xp_ref[0] + jnp.dot(h[...], whh[...],
                            preferred_element_type=jnp.float32)
    i_g = g[:, 0:DP]
    f_g = g[:, DP:2 * DP]
    g_g = g[:, 2 * DP:3 * DP]
    o_g = g[:, 3 * DP:4 * DP]
    c2 = jax.nn.sigmoid(f_g) * c[...] + jax.nn.sigmoid(i_g) * jnp.tanh(g_g)
    h2 = jax.nn.sigmoid(o_g) * jnp.tanh(c2)
    h[...] = h2
    c[...] = c2
    hist_out[0] = h2


def _k2(xproj, whh):
    return pl.pallas_call(
        _k2_fn,
        grid=(Tm1,),
        in_specs=[pl.BlockSpec((1, B, 4 * DP), lambda t: (t, 0, 0)),
                  pl.BlockSpec((DP, 4 * DP), lambda t: (0, 0))],
        out_specs=pl.BlockSpec((1, B, DP), lambda t: (t, 0, 0)),
        out_shape=jax.ShapeDtypeStruct((Tm1, B, DP), jnp.float32),
        scratch_shapes=[pltpu.VMEM((B, DP), jnp.float32),
                        pltpu.VMEM((B, DP), jnp.float32)],
    )(xproj, whh)


def _k3_fn(gq_ref, gs_ref, h_ref, u_ref, kv_ref, out_ref):
    Gv = gq_ref[0]
    Gs = gs_ref[0]
    H = h_ref[0]
    U = u_ref[0]
    dims = (((1,), (1,)), ((), ()))
    Srow = lax.dot_general(Gs, Gv, dims,
                           preferred_element_type=jnp.float32)
    tt = lax.broadcasted_iota(jnp.int32, (TP, TP), 0)
    ta = lax.broadcasted_iota(jnp.int32, (TP, TP), 1)
    valid = (ta < tt) & (tt < Tm1)
    Sm = jnp.where(valid, Srow, NEG)
    gtr = jnp.sum((Sm[:, :, None] > Sm[:, None, :]).astype(jnp.int32),
                  axis=1)
    i1 = lax.broadcasted_iota(jnp.int32, (TP, TP, TP), 1)
    i2 = lax.broadcasted_iota(jnp.int32, (TP, TP, TP), 2)
    eqc = jnp.sum(((Sm[:, :, None] == Sm[:, None, :]) & (i1 < i2))
                  .astype(jnp.int32), axis=1)
    rank = gtr + eqc
    sel = (valid & (rank < RANK_K)) | (valid & (tt < RANK_K)) \
        | ((ta == tt) & (tt < Tm1))
    hk = lax.dot_general(kv_ref[...], H, dims,
                         preferred_element_type=jnp.float32)
    hkb = jnp.broadcast_to(hk, (TP, TP))
    mk = jnp.max(jnp.where(sel, hkb, NEG), axis=1, keepdims=True)
    wk = jnp.where(sel, jnp.exp(hkb - mk), 0.0)
    sk = jnp.sum(wk, axis=1, keepdims=True)
    V = jnp.dot(wk, H, preferred_element_type=jnp.float32)
    num = jnp.sum(U * V, axis=1, keepdims=True)
    p = num / sk
    out_ref[0] = jnp.broadcast_to(p, (TP, DP))


def _k3(gqb, gsb, histb, unb, kvec):
    spec = pl.BlockSpec((1, TP, DP), lambda b: (b, 0, 0))
    return pl.pallas_call(
        _k3_fn,
        grid=(B,),
        in_specs=[spec, spec, spec, spec,
                  pl.BlockSpec((1, DP), lambda b: (0, 0))],
        out_specs=spec,
        out_shape=jax.ShapeDtypeStruct((B, TP, DP), jnp.float32),
    )(gqb, gsb, histb, unb, kvec)


def kernel(emb_q, emb_q2, emb_s, emb_u, emb_r, w1_q, w2_q, W_ih, W_hh,
           b_ih, b_hh, fusion_W, fusion_b, agg_W, agg_b, aggL_W, aggL_b,
           q_W, q_b, k_W, k_b, w_W, w_b, user, question, response, mask,
           q_neighbors, s_neighbors, u_neighbors, q_neighbors_2,
           q_skill_idx):
    f32 = jnp.float32
    padD = lambda x: jnp.pad(x, ((0, 0), (0, DP - D)))
    embq_p = padD(emb_q)
    embs_p = padD(emb_s)
    embu_p = padD(emb_u)
    embq2_p = padD(emb_q2)

    qflat = jnp.pad(question.T.reshape(-1).astype(jnp.int32),
                    (0, 3520 - T * B))
    uflat = jnp.pad(user.T.reshape(-1).astype(jnp.int32), (0, 3520 - T * B))

    qn_flat = jnp.pad(q_neighbors.astype(jnp.int32).reshape(-1),
                      (0, 4 * 10240 - 4 * NQ))
    un_flat = jnp.pad(u_neighbors.astype(jnp.int32).reshape(-1),
                      (0, 4 * 20480 - 4 * NU))
    outs = _sc_gather(qflat, uflat,
                      qn_flat,
                      s_neighbors.astype(jnp.int32).reshape(-1),
                      un_flat,
                      q_neighbors_2.astype(jnp.int32).reshape(-1),
                      q_skill_idx.astype(jnp.int32).reshape(-1),
                      embq_p, embs_p, embu_p, embq2_p)
    E0, E1, N2f, F0, F1, M2f, QN, SK, E3Q, E3U = outs
    TQ = _km(jnp.pad(embq_p, ((0, 10240 - NQ), (0, 0))), E3Q, 10240)
    TU = _km(jnp.pad(embu_p, ((0, 20480 - NU), (0, 0))), E3U, 20480)
    EM, FM = _sc2(TQ, TU, N2f, M2f)

    padW = lambda w: jnp.pad(w, ((0, DP - w.shape[0]), (0, DP - w.shape[1])))
    W0, W1, W2 = padW(agg_W[0]), padW(agg_W[1]), padW(agg_W[2])
    WL = padW(aggL_W)
    Fw1 = padW(fusion_W[:D])
    bp = jnp.zeros((8, DP), f32)
    bp = bp.at[0, :D].set(agg_b[0]).at[1, :D].set(agg_b[1])
    bp = bp.at[2, :D].set(agg_b[2]).at[3, :D].set(aggL_b)
    def pad_gates(w):
        out = jnp.zeros((DP, 4 * DP), f32)
        for g in range(4):
            out = out.at[:D, g * DP:g * DP + D].set(w[:, g * D:(g + 1) * D])
        return out
    Wih = pad_gates(W_ih)
    Whh = pad_gates(W_hh)
    xb = jnp.zeros((1, 4 * DP), f32)
    bsum = b_ih + b_hh
    for g in range(4):
        xb = xb.at[0, g * DP:g * DP + D].set(bsum[g * D:(g + 1) * D])
    v0 = emb_r[0] @ fusion_W[D:] + fusion_b
    v1 = emb_r[1] @ fusion_W[D:] + fusion_b
    rt = response.T.reshape(-1)[:N_REAL]
    er2 = jnp.where((rt[:, None] > 0), jnp.pad(v1, (0, DP - D)),
                    jnp.pad(v0, (0, DP - D)))
    er2 = jnp.pad(er2, ((0, NT - N_REAL), (0, 0)))
    wv = jnp.stack([jnp.full((DP,), w1_q, f32), jnp.full((DP,), w2_q, f32)])
    qvec = jnp.pad(q_W @ w_W[:D, 0], (0, DP - D))[None, :]
    kvec = jnp.pad(k_W @ w_W[D:, 0], (0, DP - D))[None, :]
    P1 = (jnp.kron(jnp.eye(TB, dtype=f32), jnp.ones((1, 4), f32)) * 0.25)
    P2 = (jnp.kron(jnp.eye(4 * TB, dtype=f32), jnp.ones((1, 4), f32)) * 0.25)

    xproj, u_norm = _k1(E0, E1, EM, F0, F1, FM, QN, SK, er2,
                        W0, W1, W2, WL, Fw1, Wih, P1, P2, bp, xb, wv, qvec)

    hist_t = _k2(xproj.reshape(G1, B, 4 * DP), Whh)

    Gq_t = E0[:T * B].reshape(T, B, DP)
    gqb = jnp.pad(Gq_t.transpose(1, 0, 2), ((0, 0), (0, TP - T), (0, 0)))
    gsb = jnp.pad(Gq_t[1:].transpose(1, 0, 2),
                  ((0, 0), (0, TP - Tm1), (0, 0)))
    histb_p = jnp.pad(hist_t.transpose(1, 0, 2), ((0, 0), (0, TP - Tm1), (0, 0)))
    unb = jnp.pad(u_norm[:N_REAL].reshape(Tm1, B, DP).transpose(1, 0, 2),
                  ((0, 0), (0, TP - Tm1), (0, 0)))

    P = _k3(gqb, gsb, histb_p, unb, kvec)
    p = jax.nn.sigmoid(P[:, :Tm1, 0])
    return jnp.concatenate([jnp.zeros((B, 1), f32), p], axis=1)
